# Initial kernel scaffold; baseline (speedup 1.0000x reference)
import jax
import jax.numpy as jnp
from jax.experimental import pallas as pl


def kernel(nf, edge_index_out, nef_out, edge_index_in, nef_in, w1_o2i, b1_o2i, w2_o2i, b2_o2i, w1_i2o, b1_i2o, w2_i2o, b2_i2o, w1_red, b1_red, w2_red, b2_red):
    raise NotImplementedError("write your pallas kernel here")


# speedup vs baseline: 2.9687x; 2.9687x over previous
import functools
import jax
import jax.numpy as jnp
from jax import lax
from jax.experimental import pallas as pl
from jax.experimental.pallas import tpu as pltpu
from jax.experimental.pallas import tpu_sc as plsc

N = 10000
E = 320000
D = 128
HIN = 16

NC = 2
NS = 16
NW = NC * NS
EPW = E // NW
CHUNK = 80
NCHUNK = EPW // CHUNK
RPS = 624
TAIL = N - NS * RPS


def _proj_body(nf_ref, w_ref, out_ref):
    out_ref[...] = jnp.dot(nf_ref[...], w_ref[...],
                           preferred_element_type=jnp.float32)


def _edge_bias_body(nefo_ref, nefi_ref, wo_ref, bo_ref, wi_ref, bi_ref,
                    co_ref, ci_ref):
    co_ref[...] = jnp.dot(nefo_ref[...], wo_ref[...],
                          preferred_element_type=jnp.float32) + bo_ref[...]
    ci_ref[...] = jnp.dot(nefi_ref[...], wi_ref[...],
                          preferred_element_type=jnp.float32) + bi_ref[...]


def _gate_body(h_ref, wk_ref, bk_ref, wg_ref, bg_ref, g_ref):
    h = h_ref[...]
    m0 = jnp.sum(h * wk_ref[...], axis=1, keepdims=True) + bk_ref[...]
    k = jax.nn.sigmoid(m0)
    g_ref[...] = (jnp.dot(h, wg_ref[...],
                          preferred_element_type=jnp.float32)
                  + bg_ref[...]) * k


def _final_body(sop_ref, dgo_ref, dgi_ref, tg_ref,
                w2o_ref, b2o_ref, w1a_ref, w1b_ref, b1r_ref,
                w2r_ref, b2r_ref, out_ref):
    hsum = sop_ref[0] + sop_ref[1]
    dego = dgo_ref[0][:, 0:1] + dgo_ref[1][:, 0:1]
    degi = dgi_ref[0][:, 0:1] + dgi_ref[1][:, 0:1]
    new_nf = (jnp.dot(hsum, w2o_ref[...],
                      preferred_element_type=jnp.float32)
              + dego * b2o_ref[...])
    t = tg_ref[0] + tg_ref[1]
    lane = lax.broadcasted_iota(jnp.int32, t.shape, 1)
    scale = jnp.where(lane < 8, 1.0, 1.0 / jnp.maximum(degi, 1.0))
    ts = t * scale
    hr = (jnp.dot(new_nf, w1a_ref[...], preferred_element_type=jnp.float32)
          + jnp.dot(ts, w1b_ref[...], preferred_element_type=jnp.float32)
          + b1r_ref[...])
    hr = jnp.maximum(hr, 0.2 * hr)
    red = jnp.dot(hr, w2r_ref[...],
                  preferred_element_type=jnp.float32) + b2r_ref[...]
    out_ref[...] = jnp.where(degi > 0, red, new_nf)


def _sc_edges_body(ao, bo, co, srco, dsto, ai, bi, ci, srci, dsti,
                   sop, dgo, dgi, hout,
                   src_idx, dst_idx, a_buf, b_buf, c_buf, h_buf,
                   ones_buf, stage, s_acc, do_acc, di_acc,
                   sem_a, sem_b, sem_c):
    cid = lax.axis_index("c")
    sid = lax.axis_index("s")
    wid = sid * NC + cid

    def _zrow(i, _):
        stage[i, :] = jnp.zeros((16,), jnp.float32)
        return 0
    lax.fori_loop(0, RPS, _zrow, 0)
    r0 = sid * RPS
    pltpu.sync_copy(stage, s_acc.at[pl.ds(r0, RPS)])
    pltpu.sync_copy(stage, do_acc.at[pl.ds(r0, RPS)])
    pltpu.sync_copy(stage, di_acc.at[pl.ds(r0, RPS)])

    @pl.when(sid == NS - 1)
    def _zero_tail():
        tail0 = NS * RPS
        pltpu.sync_copy(stage.at[pl.ds(0, TAIL)], s_acc.at[pl.ds(tail0, TAIL)])
        pltpu.sync_copy(stage.at[pl.ds(0, TAIL)], do_acc.at[pl.ds(tail0, TAIL)])
        pltpu.sync_copy(stage.at[pl.ds(0, TAIL)], di_acc.at[pl.ds(tail0, TAIL)])

    onehot = jnp.where(lax.iota(jnp.int32, 16) == 0, 1.0, 0.0)

    def _orow(i, _):
        ones_buf[i, :] = onehot
        return 0
    lax.fori_loop(0, CHUNK, _orow, 0)
    plsc.subcore_barrier()

    base0 = wid * EPW

    def _edge_pass(g, src_hbm, dst_hbm, a_tab, b_tab, c_hbm):
        base = base0 + g * CHUNK
        pltpu.sync_copy(src_hbm.at[pl.ds(base, CHUNK)], src_idx)
        pltpu.sync_copy(dst_hbm.at[pl.ds(base, CHUNK)], dst_idx)
        cp_a = pltpu.async_copy(a_tab.at[src_idx], a_buf, sem_a)
        cp_b = pltpu.async_copy(b_tab.at[dst_idx], b_buf, sem_b)
        cp_c = pltpu.async_copy(c_hbm.at[pl.ds(base, CHUNK)], c_buf, sem_c)
        cp_a.wait()
        cp_b.wait()
        cp_c.wait()

        def _row(i, _):
            h = a_buf[i, :] + b_buf[i, :] + c_buf[i, :]
            h_buf[i, :] = jnp.maximum(h, 0.2 * h)
            return 0
        lax.fori_loop(0, CHUNK, _row, 0)
        return base

    def _o2i(g, _):
        _edge_pass(g, srco, dsto, ao, bo, co)
        pltpu.sync_copy(h_buf, s_acc.at[dst_idx], add=True)
        pltpu.sync_copy(ones_buf, do_acc.at[dst_idx], add=True)
        return 0
    lax.fori_loop(0, NCHUNK, _o2i, 0)

    def _i2o(g, _):
        base = _edge_pass(g, srci, dsti, ai, bi, ci)
        pltpu.sync_copy(h_buf, hout.at[pl.ds(base, CHUNK)])
        pltpu.sync_copy(ones_buf, di_acc.at[dst_idx], add=True)
        return 0
    lax.fori_loop(0, NCHUNK, _i2o, 0)

    plsc.subcore_barrier()

    pltpu.sync_copy(s_acc.at[pl.ds(r0, RPS)], stage)
    pltpu.sync_copy(stage, sop.at[cid, pl.ds(r0, RPS)])
    pltpu.sync_copy(do_acc.at[pl.ds(r0, RPS)], stage)
    pltpu.sync_copy(stage, dgo.at[cid, pl.ds(r0, RPS)])
    pltpu.sync_copy(di_acc.at[pl.ds(r0, RPS)], stage)
    pltpu.sync_copy(stage, dgi.at[cid, pl.ds(r0, RPS)])

    @pl.when(sid == NS - 1)
    def _out_tail():
        tail0 = NS * RPS
        for acc, dst in ((s_acc, sop), (do_acc, dgo), (di_acc, dgi)):
            pltpu.sync_copy(acc.at[pl.ds(tail0, TAIL)],
                            stage.at[pl.ds(0, TAIL)])
            pltpu.sync_copy(stage.at[pl.ds(0, TAIL)],
                            dst.at[cid, pl.ds(tail0, TAIL)])


def _sc_scatter_body(g_hbm, dsti, tg,
                     dst_idx, g_buf, stage, t_acc, sem_g):
    cid = lax.axis_index("c")
    sid = lax.axis_index("s")
    wid = sid * NC + cid

    def _zrow(i, _):
        stage[i, :] = jnp.zeros((16,), jnp.float32)
        return 0
    lax.fori_loop(0, RPS, _zrow, 0)
    r0 = sid * RPS
    pltpu.sync_copy(stage, t_acc.at[pl.ds(r0, RPS)])

    @pl.when(sid == NS - 1)
    def _zero_tail():
        pltpu.sync_copy(stage.at[pl.ds(0, TAIL)],
                        t_acc.at[pl.ds(NS * RPS, TAIL)])
    plsc.subcore_barrier()

    base0 = wid * EPW

    def _chunk(g, _):
        base = base0 + g * CHUNK
        pltpu.sync_copy(dsti.at[pl.ds(base, CHUNK)], dst_idx)
        cp = pltpu.async_copy(g_hbm.at[pl.ds(base, CHUNK)], g_buf, sem_g)
        cp.wait()
        pltpu.sync_copy(g_buf, t_acc.at[dst_idx], add=True)
        return 0
    lax.fori_loop(0, NCHUNK, _chunk, 0)

    plsc.subcore_barrier()
    pltpu.sync_copy(t_acc.at[pl.ds(r0, RPS)], stage)
    pltpu.sync_copy(stage, tg.at[cid, pl.ds(r0, RPS)])

    @pl.when(sid == NS - 1)
    def _out_tail():
        tail0 = NS * RPS
        pltpu.sync_copy(t_acc.at[pl.ds(tail0, TAIL)], stage.at[pl.ds(0, TAIL)])
        pltpu.sync_copy(stage.at[pl.ds(0, TAIL)], tg.at[cid, pl.ds(tail0, TAIL)])


_SC_MESH = plsc.VectorSubcoreMesh(core_axis_name="c", subcore_axis_name="s")
_SC_PARAMS = pltpu.CompilerParams(use_tc_tiling_on_sc=False)

_sc_edges = pl.kernel(
    _sc_edges_body,
    out_type=(
        jax.ShapeDtypeStruct((NC, N, 16), jnp.float32),
        jax.ShapeDtypeStruct((NC, N, 16), jnp.float32),
        jax.ShapeDtypeStruct((NC, N, 16), jnp.float32),
        jax.ShapeDtypeStruct((E, 16), jnp.float32),
    ),
    mesh=_SC_MESH,
    scratch_types=[
        pltpu.VMEM((CHUNK,), jnp.int32),
        pltpu.VMEM((CHUNK,), jnp.int32),
        pltpu.VMEM((CHUNK, 16), jnp.float32),
        pltpu.VMEM((CHUNK, 16), jnp.float32),
        pltpu.VMEM((CHUNK, 16), jnp.float32),
        pltpu.VMEM((CHUNK, 16), jnp.float32),
        pltpu.VMEM((CHUNK, 16), jnp.float32),
        pltpu.VMEM((RPS, 16), jnp.float32),
        pltpu.VMEM_SHARED((N, 16), jnp.float32),
        pltpu.VMEM_SHARED((N, 16), jnp.float32),
        pltpu.VMEM_SHARED((N, 16), jnp.float32),
        pltpu.SemaphoreType.DMA,
        pltpu.SemaphoreType.DMA,
        pltpu.SemaphoreType.DMA,
    ],
    compiler_params=_SC_PARAMS,
)

_sc_scatter = pl.kernel(
    _sc_scatter_body,
    out_type=jax.ShapeDtypeStruct((NC, N, 16), jnp.float32),
    mesh=_SC_MESH,
    scratch_types=[
        pltpu.VMEM((CHUNK,), jnp.int32),
        pltpu.VMEM((CHUNK, 16), jnp.float32),
        pltpu.VMEM((RPS, 16), jnp.float32),
        pltpu.VMEM_SHARED((N, 16), jnp.float32),
        pltpu.SemaphoreType.DMA,
    ],
    compiler_params=_SC_PARAMS,
)


EB = 8000


@jax.jit
def kernel(nf, edge_index_out, nef_out, edge_index_in, nef_in,
           w1_o2i, b1_o2i, w2_o2i, b2_o2i,
           w1_i2o, b1_i2o, w2_i2o, b2_i2o,
           w1_red, b1_red, w2_red, b2_red):
    nf = nf.astype(jnp.float32)
    src_o, dst_o = edge_index_out[0], edge_index_out[1]
    src_i, dst_i = edge_index_in[0], edge_index_in[1]

    wcat = jnp.concatenate([w1_o2i[:D], w1_o2i[D:2 * D],
                            w1_i2o[:D], w1_i2o[D:2 * D]], axis=1)
    proj = pl.pallas_call(
        _proj_body,
        out_shape=jax.ShapeDtypeStruct((N, 64), jnp.float32),
    )(nf, wcat)
    ao, bo = proj[:, 0:16], proj[:, 16:32]
    ai, bi = proj[:, 32:48], proj[:, 48:64]

    co, ci = pl.pallas_call(
        _edge_bias_body,
        grid=(E // EB,),
        in_specs=[
            pl.BlockSpec((EB, 16), lambda e: (e, 0)),
            pl.BlockSpec((EB, 16), lambda e: (e, 0)),
            pl.BlockSpec((16, 16), lambda e: (0, 0)),
            pl.BlockSpec((1, 16), lambda e: (0, 0)),
            pl.BlockSpec((16, 16), lambda e: (0, 0)),
            pl.BlockSpec((1, 16), lambda e: (0, 0)),
        ],
        out_specs=[
            pl.BlockSpec((EB, 16), lambda e: (e, 0)),
            pl.BlockSpec((EB, 16), lambda e: (e, 0)),
        ],
        out_shape=[
            jax.ShapeDtypeStruct((E, 16), jnp.float32),
            jax.ShapeDtypeStruct((E, 16), jnp.float32),
        ],
    )(nef_out, nef_in,
      w1_o2i[2 * D:], b1_o2i.reshape(1, 16),
      w1_i2o[2 * D:], b1_i2o.reshape(1, 16))

    sop, dgo, dgi, h_i = _sc_edges(ao, bo, co, src_o, dst_o,
                                   ai, bi, ci, src_i, dst_i)

    g = pl.pallas_call(
        _gate_body,
        grid=(E // EB,),
        in_specs=[
            pl.BlockSpec((EB, 16), lambda e: (e, 0)),
            pl.BlockSpec((1, 16), lambda e: (0, 0)),
            pl.BlockSpec((1, 1), lambda e: (0, 0)),
            pl.BlockSpec((16, 16), lambda e: (0, 0)),
            pl.BlockSpec((1, 16), lambda e: (0, 0)),
        ],
        out_specs=pl.BlockSpec((EB, 16), lambda e: (e, 0)),
        out_shape=jax.ShapeDtypeStruct((E, 16), jnp.float32),
    )(h_i, w2_i2o[:, 0].reshape(1, 16), b2_i2o[0].reshape(1, 1),
      w2_i2o[:, 1:], b2_i2o[1:].reshape(1, 16))

    tg = _sc_scatter(g, dst_i)

    nb = 2000
    out = pl.pallas_call(
        _final_body,
        grid=(N // nb,),
        in_specs=[
            pl.BlockSpec((NC, nb, 16), lambda n: (0, n, 0)),
            pl.BlockSpec((NC, nb, 16), lambda n: (0, n, 0)),
            pl.BlockSpec((NC, nb, 16), lambda n: (0, n, 0)),
            pl.BlockSpec((NC, nb, 16), lambda n: (0, n, 0)),
            pl.BlockSpec((16, D), lambda n: (0, 0)),
            pl.BlockSpec((1, D), lambda n: (0, 0)),
            pl.BlockSpec((D, HIN), lambda n: (0, 0)),
            pl.BlockSpec((16, HIN), lambda n: (0, 0)),
            pl.BlockSpec((1, HIN), lambda n: (0, 0)),
            pl.BlockSpec((HIN, D), lambda n: (0, 0)),
            pl.BlockSpec((1, D), lambda n: (0, 0)),
        ],
        out_specs=pl.BlockSpec((nb, D), lambda n: (n, 0)),
        out_shape=jax.ShapeDtypeStruct((N, D), jnp.float32),
    )(sop, dgo, dgi, tg,
      w2_o2i, b2_o2i.reshape(1, D),
      w1_red[:D], w1_red[D:], b1_red.reshape(1, HIN),
      w2_red, b2_red.reshape(1, D))
    return out


# speedup vs baseline: 4.6802x; 1.5765x over previous
import functools
import jax
import jax.numpy as jnp
from jax import lax
from jax.experimental import pallas as pl
from jax.experimental.pallas import tpu as pltpu
from jax.experimental.pallas import tpu_sc as plsc

N = 10000
E = 320000
D = 128
HIN = 16

NC = 2
NS = 16
NW = NC * NS
EPW = E // NW
CHUNK = 80
NCHUNK = EPW // CHUNK
RPS = 624
TAIL = N - NS * RPS


def _proj_body(nf_ref, w_ref, out_ref):
    out_ref[...] = jnp.dot(nf_ref[...], w_ref[...],
                           preferred_element_type=jnp.float32)


def _edge_bias_body(nefo_ref, nefi_ref, wo_ref, bo_ref, wi_ref, bi_ref,
                    co_ref, ci_ref):
    co_ref[...] = jnp.dot(nefo_ref[...], wo_ref[...],
                          preferred_element_type=jnp.float32) + bo_ref[...]
    ci_ref[...] = jnp.dot(nefi_ref[...], wi_ref[...],
                          preferred_element_type=jnp.float32) + bi_ref[...]


def _gate_body(h_ref, wk_ref, bk_ref, wg_ref, bg_ref, g_ref):
    h = h_ref[...]
    m0 = jnp.dot(h, wk_ref[...], preferred_element_type=jnp.float32)
    k = jax.nn.sigmoid(m0 + bk_ref[...])
    g_ref[...] = (jnp.dot(h, wg_ref[...],
                          preferred_element_type=jnp.float32)
                  + bg_ref[...]) * k


def _final_body(sop_ref, dgo_ref, dgi_ref, tg_ref,
                w2o_ref, b2o_ref, w1a_ref, w1b_ref, b1r_ref,
                w2r_ref, b2r_ref, out_ref):
    hsum = sop_ref[0] + sop_ref[1]
    dego = dgo_ref[0][:, 0:1] + dgo_ref[1][:, 0:1]
    degi = dgi_ref[0][:, 0:1] + dgi_ref[1][:, 0:1]
    new_nf = (jnp.dot(hsum, w2o_ref[...],
                      preferred_element_type=jnp.float32)
              + dego * b2o_ref[...])
    t = tg_ref[0] + tg_ref[1]
    lane = lax.broadcasted_iota(jnp.int32, t.shape, 1)
    scale = jnp.where(lane < 8, 1.0, 1.0 / jnp.maximum(degi, 1.0))
    ts = t * scale
    hr = (jnp.dot(new_nf, w1a_ref[...], preferred_element_type=jnp.float32)
          + jnp.dot(ts, w1b_ref[...], preferred_element_type=jnp.float32)
          + b1r_ref[...])
    hr = jnp.maximum(hr, 0.2 * hr)
    red = jnp.dot(hr, w2r_ref[...],
                  preferred_element_type=jnp.float32) + b2r_ref[...]
    out_ref[...] = jnp.where(degi > 0, red, new_nf)


def _sc_edges_body(ao, bo, co, srco, dsto, ai, bi, ci, srci, dsti,
                   sop, dgo, dgi, hout,
                   src_idx, dst_idx, a_buf, b_buf, c_buf, h_buf,
                   ones_buf, stage, s_acc, do_acc, di_acc,
                   sem_a, sem_b, sem_c):
    cid = lax.axis_index("c")
    sid = lax.axis_index("s")
    wid = sid * NC + cid

    def _zrow(i, _):
        stage[i, :] = jnp.zeros((16,), jnp.float32)
        return 0
    lax.fori_loop(0, RPS, _zrow, 0)
    r0 = sid * RPS
    pltpu.sync_copy(stage, s_acc.at[pl.ds(r0, RPS)])
    pltpu.sync_copy(stage, do_acc.at[pl.ds(r0, RPS)])
    pltpu.sync_copy(stage, di_acc.at[pl.ds(r0, RPS)])

    @pl.when(sid == NS - 1)
    def _zero_tail():
        tail0 = NS * RPS
        pltpu.sync_copy(stage.at[pl.ds(0, TAIL)], s_acc.at[pl.ds(tail0, TAIL)])
        pltpu.sync_copy(stage.at[pl.ds(0, TAIL)], do_acc.at[pl.ds(tail0, TAIL)])
        pltpu.sync_copy(stage.at[pl.ds(0, TAIL)], di_acc.at[pl.ds(tail0, TAIL)])

    onehot = jnp.where(lax.iota(jnp.int32, 16) == 0, 1.0, 0.0)

    def _orow(i, _):
        ones_buf[i, :] = onehot
        return 0
    lax.fori_loop(0, CHUNK, _orow, 0)
    plsc.subcore_barrier()

    base0 = wid * EPW

    def _edge_pass(g, src_hbm, dst_hbm, a_tab, b_tab, c_hbm):
        base = base0 + g * CHUNK
        pltpu.sync_copy(src_hbm.at[pl.ds(base, CHUNK)], src_idx)
        pltpu.sync_copy(dst_hbm.at[pl.ds(base, CHUNK)], dst_idx)
        cp_a = pltpu.async_copy(a_tab.at[src_idx], a_buf, sem_a)
        cp_b = pltpu.async_copy(b_tab.at[dst_idx], b_buf, sem_b)
        cp_c = pltpu.async_copy(c_hbm.at[pl.ds(base, CHUNK)], c_buf, sem_c)
        cp_a.wait()
        cp_b.wait()
        cp_c.wait()

        def _row(i, _):
            h = a_buf[i, :] + b_buf[i, :] + c_buf[i, :]
            h_buf[i, :] = jnp.maximum(h, 0.2 * h)
            return 0
        lax.fori_loop(0, CHUNK, _row, 0)
        return base

    def _o2i(g, _):
        _edge_pass(g, srco, dsto, ao, bo, co)
        pltpu.sync_copy(h_buf, s_acc.at[dst_idx], add=True)
        pltpu.sync_copy(ones_buf, do_acc.at[dst_idx], add=True)
        return 0
    lax.fori_loop(0, NCHUNK, _o2i, 0)

    def _i2o(g, _):
        base = _edge_pass(g, srci, dsti, ai, bi, ci)
        pltpu.sync_copy(h_buf, hout.at[pl.ds(base, CHUNK)])
        pltpu.sync_copy(ones_buf, di_acc.at[dst_idx], add=True)
        return 0
    lax.fori_loop(0, NCHUNK, _i2o, 0)

    plsc.subcore_barrier()

    pltpu.sync_copy(s_acc.at[pl.ds(r0, RPS)], stage)
    pltpu.sync_copy(stage, sop.at[cid, pl.ds(r0, RPS)])
    pltpu.sync_copy(do_acc.at[pl.ds(r0, RPS)], stage)
    pltpu.sync_copy(stage, dgo.at[cid, pl.ds(r0, RPS)])
    pltpu.sync_copy(di_acc.at[pl.ds(r0, RPS)], stage)
    pltpu.sync_copy(stage, dgi.at[cid, pl.ds(r0, RPS)])

    @pl.when(sid == NS - 1)
    def _out_tail():
        tail0 = NS * RPS
        for acc, dst in ((s_acc, sop), (do_acc, dgo), (di_acc, dgi)):
            pltpu.sync_copy(acc.at[pl.ds(tail0, TAIL)],
                            stage.at[pl.ds(0, TAIL)])
            pltpu.sync_copy(stage.at[pl.ds(0, TAIL)],
                            dst.at[cid, pl.ds(tail0, TAIL)])


def _sc_scatter_body(g_hbm, dsti, tg,
                     dst_idx, g_buf, stage, t_acc, sem_g):
    cid = lax.axis_index("c")
    sid = lax.axis_index("s")
    wid = sid * NC + cid

    def _zrow(i, _):
        stage[i, :] = jnp.zeros((16,), jnp.float32)
        return 0
    lax.fori_loop(0, RPS, _zrow, 0)
    r0 = sid * RPS
    pltpu.sync_copy(stage, t_acc.at[pl.ds(r0, RPS)])

    @pl.when(sid == NS - 1)
    def _zero_tail():
        pltpu.sync_copy(stage.at[pl.ds(0, TAIL)],
                        t_acc.at[pl.ds(NS * RPS, TAIL)])
    plsc.subcore_barrier()

    base0 = wid * EPW

    def _chunk(g, _):
        base = base0 + g * CHUNK
        pltpu.sync_copy(dsti.at[pl.ds(base, CHUNK)], dst_idx)
        cp = pltpu.async_copy(g_hbm.at[pl.ds(base, CHUNK)], g_buf, sem_g)
        cp.wait()
        pltpu.sync_copy(g_buf, t_acc.at[dst_idx], add=True)
        return 0
    lax.fori_loop(0, NCHUNK, _chunk, 0)

    plsc.subcore_barrier()
    pltpu.sync_copy(t_acc.at[pl.ds(r0, RPS)], stage)
    pltpu.sync_copy(stage, tg.at[cid, pl.ds(r0, RPS)])

    @pl.when(sid == NS - 1)
    def _out_tail():
        tail0 = NS * RPS
        pltpu.sync_copy(t_acc.at[pl.ds(tail0, TAIL)], stage.at[pl.ds(0, TAIL)])
        pltpu.sync_copy(stage.at[pl.ds(0, TAIL)], tg.at[cid, pl.ds(tail0, TAIL)])


_SC_MESH = plsc.VectorSubcoreMesh(core_axis_name="c", subcore_axis_name="s")
_SC_PARAMS = pltpu.CompilerParams(use_tc_tiling_on_sc=False)

_sc_edges = pl.kernel(
    _sc_edges_body,
    out_type=(
        jax.ShapeDtypeStruct((NC, N, 16), jnp.float32),
        jax.ShapeDtypeStruct((NC, N, 16), jnp.float32),
        jax.ShapeDtypeStruct((NC, N, 16), jnp.float32),
        jax.ShapeDtypeStruct((E, 16), jnp.float32),
    ),
    mesh=_SC_MESH,
    scratch_types=[
        pltpu.VMEM((CHUNK,), jnp.int32),
        pltpu.VMEM((CHUNK,), jnp.int32),
        pltpu.VMEM((CHUNK, 16), jnp.float32),
        pltpu.VMEM((CHUNK, 16), jnp.float32),
        pltpu.VMEM((CHUNK, 16), jnp.float32),
        pltpu.VMEM((CHUNK, 16), jnp.float32),
        pltpu.VMEM((CHUNK, 16), jnp.float32),
        pltpu.VMEM((RPS, 16), jnp.float32),
        pltpu.VMEM_SHARED((N, 16), jnp.float32),
        pltpu.VMEM_SHARED((N, 16), jnp.float32),
        pltpu.VMEM_SHARED((N, 16), jnp.float32),
        pltpu.SemaphoreType.DMA,
        pltpu.SemaphoreType.DMA,
        pltpu.SemaphoreType.DMA,
    ],
    compiler_params=_SC_PARAMS,
)

_sc_scatter = pl.kernel(
    _sc_scatter_body,
    out_type=jax.ShapeDtypeStruct((NC, N, 16), jnp.float32),
    mesh=_SC_MESH,
    scratch_types=[
        pltpu.VMEM((CHUNK,), jnp.int32),
        pltpu.VMEM((CHUNK, 16), jnp.float32),
        pltpu.VMEM((RPS, 16), jnp.float32),
        pltpu.VMEM_SHARED((N, 16), jnp.float32),
        pltpu.SemaphoreType.DMA,
    ],
    compiler_params=_SC_PARAMS,
)


E8 = E // 8
EB8 = 5000


@jax.jit
def kernel(nf, edge_index_out, nef_out, edge_index_in, nef_in,
           w1_o2i, b1_o2i, w2_o2i, b2_o2i,
           w1_i2o, b1_i2o, w2_i2o, b2_i2o,
           w1_red, b1_red, w2_red, b2_red):
    nf = nf.astype(jnp.float32)
    src_o, dst_o = edge_index_out[0], edge_index_out[1]
    src_i, dst_i = edge_index_in[0], edge_index_in[1]

    wcat = jnp.concatenate([w1_o2i[:D], w1_o2i[D:2 * D],
                            w1_i2o[:D], w1_i2o[D:2 * D]], axis=1)
    proj = pl.pallas_call(
        _proj_body,
        out_shape=jax.ShapeDtypeStruct((N, 64), jnp.float32),
    )(nf, wcat)
    ao, bo = proj[:, 0:16], proj[:, 16:32]
    ai, bi = proj[:, 32:48], proj[:, 48:64]

    eye8 = jnp.eye(8, dtype=jnp.float32)
    co8, ci8 = pl.pallas_call(
        _edge_bias_body,
        grid=(E8 // EB8,),
        in_specs=[
            pl.BlockSpec((EB8, D), lambda e: (e, 0)),
            pl.BlockSpec((EB8, D), lambda e: (e, 0)),
            pl.BlockSpec((D, D), lambda e: (0, 0)),
            pl.BlockSpec((1, D), lambda e: (0, 0)),
            pl.BlockSpec((D, D), lambda e: (0, 0)),
            pl.BlockSpec((1, D), lambda e: (0, 0)),
        ],
        out_specs=[
            pl.BlockSpec((EB8, D), lambda e: (e, 0)),
            pl.BlockSpec((EB8, D), lambda e: (e, 0)),
        ],
        out_shape=[
            jax.ShapeDtypeStruct((E8, D), jnp.float32),
            jax.ShapeDtypeStruct((E8, D), jnp.float32),
        ],
    )(nef_out.reshape(E8, D), nef_in.reshape(E8, D),
      jnp.kron(eye8, w1_o2i[2 * D:]), jnp.tile(b1_o2i, 8).reshape(1, D),
      jnp.kron(eye8, w1_i2o[2 * D:]), jnp.tile(b1_i2o, 8).reshape(1, D))
    co = co8.reshape(E, 16)
    ci = ci8.reshape(E, 16)

    sop, dgo, dgi, h_i = _sc_edges(ao, bo, co, src_o, dst_o,
                                   ai, bi, ci, src_i, dst_i)

    wk8 = jnp.kron(eye8, w2_i2o[:, 0:1] * jnp.ones((1, 16), jnp.float32))
    g8 = pl.pallas_call(
        _gate_body,
        grid=(E8 // EB8,),
        in_specs=[
            pl.BlockSpec((EB8, D), lambda e: (e, 0)),
            pl.BlockSpec((D, D), lambda e: (0, 0)),
            pl.BlockSpec((1, 1), lambda e: (0, 0)),
            pl.BlockSpec((D, D), lambda e: (0, 0)),
            pl.BlockSpec((1, D), lambda e: (0, 0)),
        ],
        out_specs=pl.BlockSpec((EB8, D), lambda e: (e, 0)),
        out_shape=jax.ShapeDtypeStruct((E8, D), jnp.float32),
    )(h_i.reshape(E8, D), wk8, b2_i2o[0].reshape(1, 1),
      jnp.kron(eye8, w2_i2o[:, 1:]), jnp.tile(b2_i2o[1:], 8).reshape(1, D))
    g = g8.reshape(E, 16)

    tg = _sc_scatter(g, dst_i)

    nb = 2000
    out = pl.pallas_call(
        _final_body,
        grid=(N // nb,),
        in_specs=[
            pl.BlockSpec((NC, nb, 16), lambda n: (0, n, 0)),
            pl.BlockSpec((NC, nb, 16), lambda n: (0, n, 0)),
            pl.BlockSpec((NC, nb, 16), lambda n: (0, n, 0)),
            pl.BlockSpec((NC, nb, 16), lambda n: (0, n, 0)),
            pl.BlockSpec((16, D), lambda n: (0, 0)),
            pl.BlockSpec((1, D), lambda n: (0, 0)),
            pl.BlockSpec((D, HIN), lambda n: (0, 0)),
            pl.BlockSpec((16, HIN), lambda n: (0, 0)),
            pl.BlockSpec((1, HIN), lambda n: (0, 0)),
            pl.BlockSpec((HIN, D), lambda n: (0, 0)),
            pl.BlockSpec((1, D), lambda n: (0, 0)),
        ],
        out_specs=pl.BlockSpec((nb, D), lambda n: (n, 0)),
        out_shape=jax.ShapeDtypeStruct((N, D), jnp.float32),
    )(sop, dgo, dgi, tg,
      w2_o2i, b2_o2i.reshape(1, D),
      w1_red[:D], w1_red[D:], b1_red.reshape(1, HIN),
      w2_red, b2_red.reshape(1, D))
    return out


# speedup vs baseline: 6.8005x; 1.4530x over previous
import functools
import jax
import jax.numpy as jnp
from jax import lax
from jax.experimental import pallas as pl
from jax.experimental.pallas import tpu as pltpu
from jax.experimental.pallas import tpu_sc as plsc

N = 10000
E = 320000
D = 128
HIN = 16

NC = 2
NS = 16
NW = NC * NS
EPW = E // NW
CHUNK = 80
NCHUNK = EPW // CHUNK
G = 5
RPS = 624
TAIL = N - NS * RPS


def _proj_body(nf_ref, w_ref, out_ref):
    out_ref[...] = jnp.dot(nf_ref[...], w_ref[...],
                           preferred_element_type=jnp.float32)


def _edge_bias_body(nefo_ref, nefi_ref, wo_ref, bo_ref, wi_ref, bi_ref,
                    co_ref, ci_ref):
    co_ref[...] = jnp.dot(nefo_ref[...], wo_ref[...],
                          preferred_element_type=jnp.float32) + bo_ref[...]
    ci_ref[...] = jnp.dot(nefi_ref[...], wi_ref[...],
                          preferred_element_type=jnp.float32) + bi_ref[...]


def _gate_body(h_ref, wk_ref, bk_ref, wg_ref, bg_ref, g_ref):
    h = h_ref[...]
    m0 = jnp.dot(h, wk_ref[...], preferred_element_type=jnp.float32)
    k = jax.nn.sigmoid(m0 + bk_ref[...])
    g_ref[...] = (jnp.dot(h, wg_ref[...],
                          preferred_element_type=jnp.float32)
                  + bg_ref[...]) * k


def _final_body(sop_ref, dgo_ref, dgi_ref, tg_ref,
                w2o_ref, b2o_ref, w1a_ref, w1b_ref, b1r_ref,
                w2r_ref, b2r_ref, out_ref):
    hsum = sop_ref[0] + sop_ref[1]
    dego = dgo_ref[0][:, 0:1] + dgo_ref[1][:, 0:1]
    degi = dgi_ref[0][:, 0:1] + dgi_ref[1][:, 0:1]
    new_nf = (jnp.dot(hsum, w2o_ref[...],
                      preferred_element_type=jnp.float32)
              + dego * b2o_ref[...])
    t = tg_ref[0] + tg_ref[1]
    lane = lax.broadcasted_iota(jnp.int32, t.shape, 1)
    scale = jnp.where(lane < 8, 1.0, 1.0 / jnp.maximum(degi, 1.0))
    ts = t * scale
    hr = (jnp.dot(new_nf, w1a_ref[...], preferred_element_type=jnp.float32)
          + jnp.dot(ts, w1b_ref[...], preferred_element_type=jnp.float32)
          + b1r_ref[...])
    hr = jnp.maximum(hr, 0.2 * hr)
    red = jnp.dot(hr, w2r_ref[...],
                  preferred_element_type=jnp.float32) + b2r_ref[...]
    out_ref[...] = jnp.where(degi > 0, red, new_nf)


def _sc_edges_body(ao, bo, co, src3o, dst3o, ai, bi, ci, src3i, dst3i,
                   sop, dgo, dgi, hout,
                   src_all, dst_all,
                   a0, a1, a2, a3, a4, b0, b1, b2, b3, b4,
                   c0, c1, c2, c3, c4, h0, h1, h2, h3, h4,
                   d0, d1, d2, d3, d4,
                   ones_buf, stage, s_acc, do_acc, di_acc,
                   gsem0, gsem1, gsem2, gsem3, gsem4, ssem, isem):
    cid = lax.axis_index("c")
    sid = lax.axis_index("s")
    wid = sid * NC + cid
    abufs = (a0, a1, a2, a3, a4)
    bbufs = (b0, b1, b2, b3, b4)
    cbufs = (c0, c1, c2, c3, c4)
    hbufs = (h0, h1, h2, h3, h4)
    dbufs = (d0, d1, d2, d3, d4)
    gsems = (gsem0, gsem1, gsem2, gsem3, gsem4)

    def _zrow(i, _):
        stage[i, :] = jnp.zeros((16,), jnp.float32)
        return 0
    lax.fori_loop(0, RPS, _zrow, 0)
    r0 = sid * RPS
    pltpu.sync_copy(stage, s_acc.at[pl.ds(r0, RPS)])
    pltpu.sync_copy(stage, do_acc.at[pl.ds(r0, RPS)])
    pltpu.sync_copy(stage, di_acc.at[pl.ds(r0, RPS)])

    @pl.when(sid == NS - 1)
    def _zero_tail():
        tail0 = NS * RPS
        pltpu.sync_copy(stage.at[pl.ds(0, TAIL)], s_acc.at[pl.ds(tail0, TAIL)])
        pltpu.sync_copy(stage.at[pl.ds(0, TAIL)], do_acc.at[pl.ds(tail0, TAIL)])
        pltpu.sync_copy(stage.at[pl.ds(0, TAIL)], di_acc.at[pl.ds(tail0, TAIL)])

    onehot = jnp.where(lax.iota(jnp.int32, 16) == 0, 1.0, 0.0)

    def _orow(i, _):
        ones_buf[i, :] = onehot
        return 0
    lax.fori_loop(0, CHUNK, _orow, 0)
    plsc.subcore_barrier()

    base0 = wid * EPW

    def _compute(s):
        def _row(i, _):
            h = abufs[s][i, :] + bbufs[s][i, :] + cbufs[s][i, :]
            hbufs[s][i, :] = jnp.maximum(h, 0.2 * h)
            return 0
        lax.fori_loop(0, CHUNK, _row, 0)

    def _run_pass(a_tab, b_tab, c_hbm, src3, dst3, o2i):
        cp0 = pltpu.async_copy(src3.at[wid], src_all, isem)
        cp1 = pltpu.async_copy(dst3.at[wid], dst_all, isem)
        cp0.wait()
        cp1.wait()

        def _group(grp, _):
            t0 = grp * G
            gcps = []
            for k in range(G):
                t = t0 + k
                gcps.append((
                    pltpu.async_copy(a_tab.at[src_all.at[t]], abufs[k],
                                     gsems[k]),
                    pltpu.async_copy(b_tab.at[dst_all.at[t]], bbufs[k],
                                     gsems[k]),
                    pltpu.async_copy(c_hbm.at[pl.ds(base0 + t * CHUNK,
                                                    CHUNK)],
                                     cbufs[k], gsems[k]),
                ))
            for k in range(G):
                t = t0 + k
                for j in range(CHUNK // 16):
                    dbufs[k][pl.ds(j * 16, 16)] = dst_all[t, pl.ds(j * 16, 16)]
                for cp in gcps[k]:
                    cp.wait()
                _compute(k)
                if o2i:
                    pltpu.sync_copy(hbufs[k], s_acc.at[dbufs[k]], add=True)
                    pltpu.sync_copy(ones_buf, do_acc.at[dbufs[k]], add=True)
                else:
                    pltpu.sync_copy(
                        hbufs[k], hout.at[pl.ds(base0 + t * CHUNK, CHUNK)])
                    pltpu.sync_copy(ones_buf, di_acc.at[dbufs[k]], add=True)
            return 0
        lax.fori_loop(0, NCHUNK // G, _group, 0)

    _run_pass(ao, bo, co, src3o, dst3o, True)
    _run_pass(ai, bi, ci, src3i, dst3i, False)

    plsc.subcore_barrier()

    pltpu.sync_copy(s_acc.at[pl.ds(r0, RPS)], stage)
    pltpu.sync_copy(stage, sop.at[cid, pl.ds(r0, RPS)])
    pltpu.sync_copy(do_acc.at[pl.ds(r0, RPS)], stage)
    pltpu.sync_copy(stage, dgo.at[cid, pl.ds(r0, RPS)])
    pltpu.sync_copy(di_acc.at[pl.ds(r0, RPS)], stage)
    pltpu.sync_copy(stage, dgi.at[cid, pl.ds(r0, RPS)])

    @pl.when(sid == NS - 1)
    def _out_tail():
        tail0 = NS * RPS
        for acc, dst in ((s_acc, sop), (do_acc, dgo), (di_acc, dgi)):
            pltpu.sync_copy(acc.at[pl.ds(tail0, TAIL)],
                            stage.at[pl.ds(0, TAIL)])
            pltpu.sync_copy(stage.at[pl.ds(0, TAIL)],
                            dst.at[cid, pl.ds(tail0, TAIL)])


def _sc_scatter_body(g_hbm, dsti, tg,
                     dst_idx, g_buf, stage, t_acc, sem_g):
    cid = lax.axis_index("c")
    sid = lax.axis_index("s")
    wid = sid * NC + cid

    def _zrow(i, _):
        stage[i, :] = jnp.zeros((16,), jnp.float32)
        return 0
    lax.fori_loop(0, RPS, _zrow, 0)
    r0 = sid * RPS
    pltpu.sync_copy(stage, t_acc.at[pl.ds(r0, RPS)])

    @pl.when(sid == NS - 1)
    def _zero_tail():
        pltpu.sync_copy(stage.at[pl.ds(0, TAIL)],
                        t_acc.at[pl.ds(NS * RPS, TAIL)])
    plsc.subcore_barrier()

    base0 = wid * EPW

    def _chunk(g, _):
        base = base0 + g * CHUNK
        pltpu.sync_copy(dsti.at[pl.ds(base, CHUNK)], dst_idx)
        cp = pltpu.async_copy(g_hbm.at[pl.ds(base, CHUNK)], g_buf, sem_g)
        cp.wait()
        pltpu.sync_copy(g_buf, t_acc.at[dst_idx], add=True)
        return 0
    lax.fori_loop(0, NCHUNK, _chunk, 0)

    plsc.subcore_barrier()
    pltpu.sync_copy(t_acc.at[pl.ds(r0, RPS)], stage)
    pltpu.sync_copy(stage, tg.at[cid, pl.ds(r0, RPS)])

    @pl.when(sid == NS - 1)
    def _out_tail():
        tail0 = NS * RPS
        pltpu.sync_copy(t_acc.at[pl.ds(tail0, TAIL)], stage.at[pl.ds(0, TAIL)])
        pltpu.sync_copy(stage.at[pl.ds(0, TAIL)], tg.at[cid, pl.ds(tail0, TAIL)])


_SC_MESH = plsc.VectorSubcoreMesh(core_axis_name="c", subcore_axis_name="s")
_SC_PARAMS = pltpu.CompilerParams(use_tc_tiling_on_sc=False)

_sc_edges = pl.kernel(
    _sc_edges_body,
    out_type=(
        jax.ShapeDtypeStruct((NC, N, 16), jnp.float32),
        jax.ShapeDtypeStruct((NC, N, 16), jnp.float32),
        jax.ShapeDtypeStruct((NC, N, 16), jnp.float32),
        jax.ShapeDtypeStruct((E, 16), jnp.float32),
    ),
    mesh=_SC_MESH,
    scratch_types=(
        [pltpu.VMEM((NCHUNK, CHUNK), jnp.int32)] * 2
        + [pltpu.VMEM((CHUNK, 16), jnp.float32)] * (4 * G)
        + [pltpu.VMEM((CHUNK,), jnp.int32)] * G
        + [pltpu.VMEM((CHUNK, 16), jnp.float32)]
        + [pltpu.VMEM((RPS, 16), jnp.float32)]
        + [pltpu.VMEM_SHARED((N, 16), jnp.float32)] * 3
        + [pltpu.SemaphoreType.DMA] * (G + 2)
    ),
    compiler_params=_SC_PARAMS,
)

_sc_scatter = pl.kernel(
    _sc_scatter_body,
    out_type=jax.ShapeDtypeStruct((NC, N, 16), jnp.float32),
    mesh=_SC_MESH,
    scratch_types=[
        pltpu.VMEM((CHUNK,), jnp.int32),
        pltpu.VMEM((CHUNK, 16), jnp.float32),
        pltpu.VMEM((RPS, 16), jnp.float32),
        pltpu.VMEM_SHARED((N, 16), jnp.float32),
        pltpu.SemaphoreType.DMA,
    ],
    compiler_params=_SC_PARAMS,
)


E8 = E // 8
EB8 = 5000


@jax.jit
def kernel(nf, edge_index_out, nef_out, edge_index_in, nef_in,
           w1_o2i, b1_o2i, w2_o2i, b2_o2i,
           w1_i2o, b1_i2o, w2_i2o, b2_i2o,
           w1_red, b1_red, w2_red, b2_red):
    nf = nf.astype(jnp.float32)
    src_o, dst_o = edge_index_out[0], edge_index_out[1]
    src_i, dst_i = edge_index_in[0], edge_index_in[1]

    wcat = jnp.concatenate([w1_o2i[:D], w1_o2i[D:2 * D],
                            w1_i2o[:D], w1_i2o[D:2 * D]], axis=1)
    proj = pl.pallas_call(
        _proj_body,
        out_shape=jax.ShapeDtypeStruct((N, 64), jnp.float32),
    )(nf, wcat)
    ao, bo = proj[:, 0:16], proj[:, 16:32]
    ai, bi = proj[:, 32:48], proj[:, 48:64]

    eye8 = jnp.eye(8, dtype=jnp.float32)
    co8, ci8 = pl.pallas_call(
        _edge_bias_body,
        grid=(E8 // EB8,),
        in_specs=[
            pl.BlockSpec((EB8, D), lambda e: (e, 0)),
            pl.BlockSpec((EB8, D), lambda e: (e, 0)),
            pl.BlockSpec((D, D), lambda e: (0, 0)),
            pl.BlockSpec((1, D), lambda e: (0, 0)),
            pl.BlockSpec((D, D), lambda e: (0, 0)),
            pl.BlockSpec((1, D), lambda e: (0, 0)),
        ],
        out_specs=[
            pl.BlockSpec((EB8, D), lambda e: (e, 0)),
            pl.BlockSpec((EB8, D), lambda e: (e, 0)),
        ],
        out_shape=[
            jax.ShapeDtypeStruct((E8, D), jnp.float32),
            jax.ShapeDtypeStruct((E8, D), jnp.float32),
        ],
    )(nef_out.reshape(E8, D), nef_in.reshape(E8, D),
      jnp.kron(eye8, w1_o2i[2 * D:]), jnp.tile(b1_o2i, 8).reshape(1, D),
      jnp.kron(eye8, w1_i2o[2 * D:]), jnp.tile(b1_i2o, 8).reshape(1, D))
    co = co8.reshape(E, 16)
    ci = ci8.reshape(E, 16)

    sop, dgo, dgi, h_i = _sc_edges(
        ao, bo, co,
        src_o.reshape(NW, NCHUNK, CHUNK), dst_o.reshape(NW, NCHUNK, CHUNK),
        ai, bi, ci,
        src_i.reshape(NW, NCHUNK, CHUNK), dst_i.reshape(NW, NCHUNK, CHUNK))

    wk8 = jnp.kron(eye8, w2_i2o[:, 0:1] * jnp.ones((1, 16), jnp.float32))
    g8 = pl.pallas_call(
        _gate_body,
        grid=(E8 // EB8,),
        in_specs=[
            pl.BlockSpec((EB8, D), lambda e: (e, 0)),
            pl.BlockSpec((D, D), lambda e: (0, 0)),
            pl.BlockSpec((1, 1), lambda e: (0, 0)),
            pl.BlockSpec((D, D), lambda e: (0, 0)),
            pl.BlockSpec((1, D), lambda e: (0, 0)),
        ],
        out_specs=pl.BlockSpec((EB8, D), lambda e: (e, 0)),
        out_shape=jax.ShapeDtypeStruct((E8, D), jnp.float32),
    )(h_i.reshape(E8, D), wk8, b2_i2o[0].reshape(1, 1),
      jnp.kron(eye8, w2_i2o[:, 1:]), jnp.tile(b2_i2o[1:], 8).reshape(1, D))
    g = g8.reshape(E, 16)

    tg = _sc_scatter(g, dst_i)

    nb = 2000
    out = pl.pallas_call(
        _final_body,
        grid=(N // nb,),
        in_specs=[
            pl.BlockSpec((NC, nb, 16), lambda n: (0, n, 0)),
            pl.BlockSpec((NC, nb, 16), lambda n: (0, n, 0)),
            pl.BlockSpec((NC, nb, 16), lambda n: (0, n, 0)),
            pl.BlockSpec((NC, nb, 16), lambda n: (0, n, 0)),
            pl.BlockSpec((16, D), lambda n: (0, 0)),
            pl.BlockSpec((1, D), lambda n: (0, 0)),
            pl.BlockSpec((D, HIN), lambda n: (0, 0)),
            pl.BlockSpec((16, HIN), lambda n: (0, 0)),
            pl.BlockSpec((1, HIN), lambda n: (0, 0)),
            pl.BlockSpec((HIN, D), lambda n: (0, 0)),
            pl.BlockSpec((1, D), lambda n: (0, 0)),
        ],
        out_specs=pl.BlockSpec((nb, D), lambda n: (n, 0)),
        out_shape=jax.ShapeDtypeStruct((N, D), jnp.float32),
    )(sop, dgo, dgi, tg,
      w2_o2i, b2_o2i.reshape(1, D),
      w1_red[:D], w1_red[D:], b1_red.reshape(1, HIN),
      w2_red, b2_red.reshape(1, D))
    return out


# speedup vs baseline: 8.1919x; 1.2046x over previous
import functools
import jax
import jax.numpy as jnp
from jax import lax
from jax.experimental import pallas as pl
from jax.experimental.pallas import tpu as pltpu
from jax.experimental.pallas import tpu_sc as plsc

N = 10000
E = 320000
D = 128
HIN = 16

NC = 2
NS = 16
NW = NC * NS
EPW = E // NW
CHUNK = 80
NCHUNK = EPW // CHUNK
G = 5
RPS = 624
TAIL = N - NS * RPS


def _proj_body(nf_ref, w_ref, out_ref):
    out_ref[...] = jnp.dot(nf_ref[...], w_ref[...],
                           preferred_element_type=jnp.float32)


def _edge_bias_body(nefo_ref, nefi_ref, wo_ref, bo_ref, wi_ref, bi_ref,
                    co_ref, ci_ref):
    co_ref[...] = jnp.dot(nefo_ref[...], wo_ref[...],
                          preferred_element_type=jnp.float32) + bo_ref[...]
    ci_ref[...] = jnp.dot(nefi_ref[...], wi_ref[...],
                          preferred_element_type=jnp.float32) + bi_ref[...]


def _gate_body(h_ref, wk_ref, bk_ref, wg_ref, bg_ref, g_ref):
    h = h_ref[...]
    m0 = jnp.dot(h, wk_ref[...], preferred_element_type=jnp.float32)
    k = jax.nn.sigmoid(m0 + bk_ref[...])
    g_ref[...] = (jnp.dot(h, wg_ref[...],
                          preferred_element_type=jnp.float32)
                  + bg_ref[...]) * k


def _final_body(sop_ref, dgo_ref, dgi_ref, tg_ref,
                w2o_ref, b2o_ref, w1a_ref, w1b_ref, b1r_ref,
                w2r_ref, b2r_ref, out_ref):
    hsum = sop_ref[0] + sop_ref[1]
    dego = dgo_ref[0][:, 0:1] + dgo_ref[1][:, 0:1]
    degi = dgi_ref[0][:, 0:1] + dgi_ref[1][:, 0:1]
    new_nf = (jnp.dot(hsum, w2o_ref[...],
                      preferred_element_type=jnp.float32)
              + dego * b2o_ref[...])
    t = tg_ref[0] + tg_ref[1]
    lane = lax.broadcasted_iota(jnp.int32, t.shape, 1)
    scale = jnp.where(lane < 8, 1.0, 1.0 / jnp.maximum(degi, 1.0))
    ts = t * scale
    hr = (jnp.dot(new_nf, w1a_ref[...], preferred_element_type=jnp.float32)
          + jnp.dot(ts, w1b_ref[...], preferred_element_type=jnp.float32)
          + b1r_ref[...])
    hr = jnp.maximum(hr, 0.2 * hr)
    red = jnp.dot(hr, w2r_ref[...],
                  preferred_element_type=jnp.float32) + b2r_ref[...]
    out_ref[...] = jnp.where(degi > 0, red, new_nf)


def _sc_pass_impl(o2i, a_tab, b_tab, c_hbm, src3, dst3, outs, hout,
                  src_all, dst_all, abufs, bbufs, cbufs, hbufs, dbufs,
                  ones_buf, stage, accs, gsems, isem):
    cid = lax.axis_index("c")
    sid = lax.axis_index("s")
    wid = sid * NC + cid

    def _zrow(i, _):
        stage[i, :] = jnp.zeros((16,), jnp.float32)
        return 0
    lax.fori_loop(0, RPS, _zrow, 0)
    r0 = sid * RPS
    for acc in accs:
        pltpu.sync_copy(stage, acc.at[pl.ds(r0, RPS)])

    @pl.when(sid == NS - 1)
    def _zero_tail():
        for acc in accs:
            pltpu.sync_copy(stage.at[pl.ds(0, TAIL)],
                            acc.at[pl.ds(NS * RPS, TAIL)])

    onehot = jnp.where(lax.iota(jnp.int32, 16) == 0, 1.0, 0.0)

    def _orow(i, _):
        ones_buf[i, :] = onehot
        return 0
    lax.fori_loop(0, CHUNK, _orow, 0)
    plsc.subcore_barrier()

    base0 = wid * EPW

    cp0 = pltpu.async_copy(src3.at[wid], src_all, isem)
    cp1 = pltpu.async_copy(dst3.at[wid], dst_all, isem)
    cp0.wait()
    cp1.wait()

    def _compute(s):
        def _row(i, _):
            h = abufs[s][i, :] + bbufs[s][i, :] + cbufs[s][i, :]
            hbufs[s][i, :] = jnp.maximum(h, 0.2 * h)
            return 0
        lax.fori_loop(0, CHUNK, _row, 0)

    def _group(grp, _):
        t0 = grp * G
        gcps = []
        for k in range(G):
            t = t0 + k
            gcps.append((
                pltpu.async_copy(a_tab.at[src_all.at[t]], abufs[k],
                                 gsems[k]),
                pltpu.async_copy(b_tab.at[dst_all.at[t]], bbufs[k],
                                 gsems[k]),
                pltpu.async_copy(c_hbm.at[pl.ds(base0 + t * CHUNK, CHUNK)],
                                 cbufs[k], gsems[k]),
            ))
        for k in range(G):
            t = t0 + k
            for j in range(CHUNK // 16):
                dbufs[k][pl.ds(j * 16, 16)] = dst_all[t, pl.ds(j * 16, 16)]
            for cp in gcps[k]:
                cp.wait()
            _compute(k)
            if o2i:
                pltpu.sync_copy(hbufs[k], accs[0].at[dbufs[k]], add=True)
                pltpu.sync_copy(ones_buf, accs[1].at[dbufs[k]], add=True)
            else:
                pltpu.sync_copy(
                    hbufs[k], hout.at[pl.ds(base0 + t * CHUNK, CHUNK)])
                pltpu.sync_copy(ones_buf, accs[0].at[dbufs[k]], add=True)
        return 0
    lax.fori_loop(0, NCHUNK // G, _group, 0)

    plsc.subcore_barrier()

    for acc, out in zip(accs, outs):
        pltpu.sync_copy(acc.at[pl.ds(r0, RPS)], stage)
        pltpu.sync_copy(stage, out.at[cid, pl.ds(r0, RPS)])

    @pl.when(sid == NS - 1)
    def _out_tail():
        tail0 = NS * RPS
        for acc, out in zip(accs, outs):
            pltpu.sync_copy(acc.at[pl.ds(tail0, TAIL)],
                            stage.at[pl.ds(0, TAIL)])
            pltpu.sync_copy(stage.at[pl.ds(0, TAIL)],
                            out.at[cid, pl.ds(tail0, TAIL)])


def _sc_o2i_body(ao, bo, co, src3, dst3, sop, dgo,
                 src_all, dst_all,
                 a0, a1, a2, a3, a4, b0, b1, b2, b3, b4,
                 c0, c1, c2, c3, c4, h0, h1, h2, h3, h4,
                 d0, d1, d2, d3, d4, ones_buf, stage, acc_h, acc_d,
                 gsem0, gsem1, gsem2, gsem3, gsem4, isem):
    _sc_pass_impl(True, ao, bo, co, src3, dst3, (sop, dgo), None,
                  src_all, dst_all,
                  (a0, a1, a2, a3, a4), (b0, b1, b2, b3, b4),
                  (c0, c1, c2, c3, c4), (h0, h1, h2, h3, h4),
                  (d0, d1, d2, d3, d4), ones_buf, stage, (acc_h, acc_d),
                  (gsem0, gsem1, gsem2, gsem3, gsem4), isem)


def _sc_i2o_body(ai, bi, ci, src3, dst3, dgi, hout,
                 src_all, dst_all,
                 a0, a1, a2, a3, a4, b0, b1, b2, b3, b4,
                 c0, c1, c2, c3, c4, h0, h1, h2, h3, h4,
                 d0, d1, d2, d3, d4, ones_buf, stage, acc_d,
                 gsem0, gsem1, gsem2, gsem3, gsem4, isem):
    _sc_pass_impl(False, ai, bi, ci, src3, dst3, (dgi,), hout,
                  src_all, dst_all,
                  (a0, a1, a2, a3, a4), (b0, b1, b2, b3, b4),
                  (c0, c1, c2, c3, c4), (h0, h1, h2, h3, h4),
                  (d0, d1, d2, d3, d4), ones_buf, stage, (acc_d,),
                  (gsem0, gsem1, gsem2, gsem3, gsem4), isem)


def _sc_scatter_body(g_hbm, dst3, tg,
                     dst_all,
                     g0, g1, g2, g3, g4, d0, d1, d2, d3, d4,
                     stage, t_acc,
                     gsem0, gsem1, gsem2, gsem3, gsem4, isem):
    cid = lax.axis_index("c")
    sid = lax.axis_index("s")
    wid = sid * NC + cid
    gbufs = (g0, g1, g2, g3, g4)
    dbufs = (d0, d1, d2, d3, d4)
    gsems = (gsem0, gsem1, gsem2, gsem3, gsem4)

    def _zrow(i, _):
        stage[i, :] = jnp.zeros((16,), jnp.float32)
        return 0
    lax.fori_loop(0, RPS, _zrow, 0)
    r0 = sid * RPS
    pltpu.sync_copy(stage, t_acc.at[pl.ds(r0, RPS)])

    @pl.when(sid == NS - 1)
    def _zero_tail():
        pltpu.sync_copy(stage.at[pl.ds(0, TAIL)],
                        t_acc.at[pl.ds(NS * RPS, TAIL)])
    plsc.subcore_barrier()

    base0 = wid * EPW
    cp = pltpu.async_copy(dst3.at[wid], dst_all, isem)
    cp.wait()

    def _group(grp, _):
        t0 = grp * G
        gcps = []
        for k in range(G):
            t = t0 + k
            gcps.append(pltpu.async_copy(
                g_hbm.at[pl.ds(base0 + t * CHUNK, CHUNK)], gbufs[k],
                gsems[k]))
        for k in range(G):
            t = t0 + k
            for j in range(CHUNK // 16):
                dbufs[k][pl.ds(j * 16, 16)] = dst_all[t, pl.ds(j * 16, 16)]
            gcps[k].wait()
            pltpu.sync_copy(gbufs[k], t_acc.at[dbufs[k]], add=True)
        return 0
    lax.fori_loop(0, NCHUNK // G, _group, 0)

    plsc.subcore_barrier()
    pltpu.sync_copy(t_acc.at[pl.ds(r0, RPS)], stage)
    pltpu.sync_copy(stage, tg.at[cid, pl.ds(r0, RPS)])

    @pl.when(sid == NS - 1)
    def _out_tail():
        tail0 = NS * RPS
        pltpu.sync_copy(t_acc.at[pl.ds(tail0, TAIL)], stage.at[pl.ds(0, TAIL)])
        pltpu.sync_copy(stage.at[pl.ds(0, TAIL)], tg.at[cid, pl.ds(tail0, TAIL)])


_SC_MESH = plsc.VectorSubcoreMesh(core_axis_name="c", subcore_axis_name="s")
_SC_PARAMS = pltpu.CompilerParams(use_tc_tiling_on_sc=False)

_EDGE_SCRATCH = (
    [pltpu.VMEM((NCHUNK, CHUNK), jnp.int32)] * 2
    + [pltpu.VMEM((CHUNK, 16), jnp.float32)] * (4 * G)
    + [pltpu.VMEM((CHUNK,), jnp.int32)] * G
    + [pltpu.VMEM((CHUNK, 16), jnp.float32)]
    + [pltpu.VMEM((RPS, 16), jnp.float32)]
)

_sc_o2i = pl.kernel(
    _sc_o2i_body,
    out_type=(
        jax.ShapeDtypeStruct((NC, N, 16), jnp.float32),
        jax.ShapeDtypeStruct((NC, N, 16), jnp.float32),
    ),
    mesh=_SC_MESH,
    scratch_types=(
        _EDGE_SCRATCH
        + [pltpu.VMEM_SHARED((N, 16), jnp.float32)] * 2
        + [pltpu.SemaphoreType.DMA] * (G + 1)
    ),
    compiler_params=_SC_PARAMS,
)

_sc_i2o = pl.kernel(
    _sc_i2o_body,
    out_type=(
        jax.ShapeDtypeStruct((NC, N, 16), jnp.float32),
        jax.ShapeDtypeStruct((E, 16), jnp.float32),
    ),
    mesh=_SC_MESH,
    scratch_types=(
        _EDGE_SCRATCH
        + [pltpu.VMEM_SHARED((N, 16), jnp.float32)]
        + [pltpu.SemaphoreType.DMA] * (G + 1)
    ),
    compiler_params=_SC_PARAMS,
)

_sc_scatter = pl.kernel(
    _sc_scatter_body,
    out_type=jax.ShapeDtypeStruct((NC, N, 16), jnp.float32),
    mesh=_SC_MESH,
    scratch_types=(
        [pltpu.VMEM((NCHUNK, CHUNK), jnp.int32)]
        + [pltpu.VMEM((CHUNK, 16), jnp.float32)] * G
        + [pltpu.VMEM((CHUNK,), jnp.int32)] * G
        + [pltpu.VMEM((RPS, 16), jnp.float32)]
        + [pltpu.VMEM_SHARED((N, 16), jnp.float32)]
        + [pltpu.SemaphoreType.DMA] * (G + 1)
    ),
    compiler_params=_SC_PARAMS,
)


E8 = E // 8
EB8 = 5000


@jax.jit
def kernel(nf, edge_index_out, nef_out, edge_index_in, nef_in,
           w1_o2i, b1_o2i, w2_o2i, b2_o2i,
           w1_i2o, b1_i2o, w2_i2o, b2_i2o,
           w1_red, b1_red, w2_red, b2_red):
    nf = nf.astype(jnp.float32)
    src_o, dst_o = edge_index_out[0], edge_index_out[1]
    src_i, dst_i = edge_index_in[0], edge_index_in[1]

    wcat = jnp.concatenate([w1_o2i[:D], w1_o2i[D:2 * D],
                            w1_i2o[:D], w1_i2o[D:2 * D]], axis=1)
    proj = pl.pallas_call(
        _proj_body,
        out_shape=jax.ShapeDtypeStruct((N, 64), jnp.float32),
    )(nf, wcat)
    ao, bo = proj[:, 0:16], proj[:, 16:32]
    ai, bi = proj[:, 32:48], proj[:, 48:64]

    eye8 = jnp.eye(8, dtype=jnp.float32)
    co8, ci8 = pl.pallas_call(
        _edge_bias_body,
        grid=(E8 // EB8,),
        in_specs=[
            pl.BlockSpec((EB8, D), lambda e: (e, 0)),
            pl.BlockSpec((EB8, D), lambda e: (e, 0)),
            pl.BlockSpec((D, D), lambda e: (0, 0)),
            pl.BlockSpec((1, D), lambda e: (0, 0)),
            pl.BlockSpec((D, D), lambda e: (0, 0)),
            pl.BlockSpec((1, D), lambda e: (0, 0)),
        ],
        out_specs=[
            pl.BlockSpec((EB8, D), lambda e: (e, 0)),
            pl.BlockSpec((EB8, D), lambda e: (e, 0)),
        ],
        out_shape=[
            jax.ShapeDtypeStruct((E8, D), jnp.float32),
            jax.ShapeDtypeStruct((E8, D), jnp.float32),
        ],
    )(nef_out.reshape(E8, D), nef_in.reshape(E8, D),
      jnp.kron(eye8, w1_o2i[2 * D:]), jnp.tile(b1_o2i, 8).reshape(1, D),
      jnp.kron(eye8, w1_i2o[2 * D:]), jnp.tile(b1_i2o, 8).reshape(1, D))
    co = co8.reshape(E, 16)
    ci = ci8.reshape(E, 16)

    sop, dgo = _sc_o2i(
        ao, bo, co,
        src_o.reshape(NW, NCHUNK, CHUNK), dst_o.reshape(NW, NCHUNK, CHUNK))
    dgi, h_i = _sc_i2o(
        ai, bi, ci,
        src_i.reshape(NW, NCHUNK, CHUNK), dst_i.reshape(NW, NCHUNK, CHUNK))

    wk8 = jnp.kron(eye8, w2_i2o[:, 0:1] * jnp.ones((1, 16), jnp.float32))
    g8 = pl.pallas_call(
        _gate_body,
        grid=(E8 // EB8,),
        in_specs=[
            pl.BlockSpec((EB8, D), lambda e: (e, 0)),
            pl.BlockSpec((D, D), lambda e: (0, 0)),
            pl.BlockSpec((1, 1), lambda e: (0, 0)),
            pl.BlockSpec((D, D), lambda e: (0, 0)),
            pl.BlockSpec((1, D), lambda e: (0, 0)),
        ],
        out_specs=pl.BlockSpec((EB8, D), lambda e: (e, 0)),
        out_shape=jax.ShapeDtypeStruct((E8, D), jnp.float32),
    )(h_i.reshape(E8, D), wk8, b2_i2o[0].reshape(1, 1),
      jnp.kron(eye8, w2_i2o[:, 1:]), jnp.tile(b2_i2o[1:], 8).reshape(1, D))
    g = g8.reshape(E, 16)

    tg = _sc_scatter(g, dst_i.reshape(NW, NCHUNK, CHUNK))

    nb = 2000
    out = pl.pallas_call(
        _final_body,
        grid=(N // nb,),
        in_specs=[
            pl.BlockSpec((NC, nb, 16), lambda n: (0, n, 0)),
            pl.BlockSpec((NC, nb, 16), lambda n: (0, n, 0)),
            pl.BlockSpec((NC, nb, 16), lambda n: (0, n, 0)),
            pl.BlockSpec((NC, nb, 16), lambda n: (0, n, 0)),
            pl.BlockSpec((16, D), lambda n: (0, 0)),
            pl.BlockSpec((1, D), lambda n: (0, 0)),
            pl.BlockSpec((D, HIN), lambda n: (0, 0)),
            pl.BlockSpec((16, HIN), lambda n: (0, 0)),
            pl.BlockSpec((1, HIN), lambda n: (0, 0)),
            pl.BlockSpec((HIN, D), lambda n: (0, 0)),
            pl.BlockSpec((1, D), lambda n: (0, 0)),
        ],
        out_specs=pl.BlockSpec((nb, D), lambda n: (n, 0)),
        out_shape=jax.ShapeDtypeStruct((N, D), jnp.float32),
    )(sop, dgo, dgi, tg,
      w2_o2i, b2_o2i.reshape(1, D),
      w1_red[:D], w1_red[D:], b1_red.reshape(1, HIN),
      w2_red, b2_red.reshape(1, D))
    return out


# speedup vs baseline: 9.2500x; 1.1292x over previous
import functools
import jax
import jax.numpy as jnp
from jax import lax
from jax.experimental import pallas as pl
from jax.experimental.pallas import tpu as pltpu
from jax.experimental.pallas import tpu_sc as plsc

N = 10000
E = 320000
D = 128
HIN = 16

NC = 2
NS = 16
NW = NC * NS
EPW = E // NW
CHUNK = 80
NCHUNK = EPW // CHUNK
G = 5
RPS = 624
TAIL = N - NS * RPS


def _proj_body(nf_ref, w_ref, out_ref):
    out_ref[...] = jnp.dot(nf_ref[...], w_ref[...],
                           preferred_element_type=jnp.float32)


def _edge_bias_body(nef_ref, w_ref, b_ref, c_ref):
    c_ref[...] = jnp.dot(nef_ref[...], w_ref[...],
                         preferred_element_type=jnp.float32) + b_ref[...]


def _gate_body(h_ref, wk_ref, bk_ref, wg_ref, bg_ref, g_ref):
    h = h_ref[...]
    m0 = jnp.dot(h, wk_ref[...], preferred_element_type=jnp.float32)
    k = jax.nn.sigmoid(m0 + bk_ref[...])
    g_ref[...] = (jnp.dot(h, wg_ref[...],
                          preferred_element_type=jnp.float32)
                  + bg_ref[...]) * k


def _final_body(sop_ref, dgo_ref, dgi_ref, tg_ref,
                w2o_ref, b2o_ref, w1a_ref, w1b_ref, b1r_ref,
                w2r_ref, b2r_ref, out_ref):
    hsum = sop_ref[0] + sop_ref[1]
    dego = dgo_ref[0][:, 0:1] + dgo_ref[1][:, 0:1]
    degi = dgi_ref[0][:, 0:1] + dgi_ref[1][:, 0:1]
    new_nf = (jnp.dot(hsum, w2o_ref[...],
                      preferred_element_type=jnp.float32)
              + dego * b2o_ref[...])
    t = tg_ref[0] + tg_ref[1]
    lane = lax.broadcasted_iota(jnp.int32, t.shape, 1)
    scale = jnp.where(lane < 8, 1.0, 1.0 / jnp.maximum(degi, 1.0))
    ts = t * scale
    hr = (jnp.dot(new_nf, w1a_ref[...], preferred_element_type=jnp.float32)
          + jnp.dot(ts, w1b_ref[...], preferred_element_type=jnp.float32)
          + b1r_ref[...])
    hr = jnp.maximum(hr, 0.2 * hr)
    red = jnp.dot(hr, w2r_ref[...],
                  preferred_element_type=jnp.float32) + b2r_ref[...]
    out_ref[...] = jnp.where(degi > 0, red, new_nf)


def _sc_pass_impl(o2i, a_tab, b_tab, c_hbm, src3, dst3, outs, hout,
                  src_all, dst_all, abufs, bbufs, cbufs, hbufs, dbufs,
                  ones_buf, stage, accs, gsems, isem):
    cid = lax.axis_index("c")
    sid = lax.axis_index("s")
    wid = sid * NC + cid

    def _zrow(i, _):
        stage[i, :] = jnp.zeros((16,), jnp.float32)
        return 0
    lax.fori_loop(0, RPS, _zrow, 0)
    r0 = sid * RPS
    for acc in accs:
        pltpu.sync_copy(stage, acc.at[pl.ds(r0, RPS)])

    @pl.when(sid == NS - 1)
    def _zero_tail():
        for acc in accs:
            pltpu.sync_copy(stage.at[pl.ds(0, TAIL)],
                            acc.at[pl.ds(NS * RPS, TAIL)])

    onehot = jnp.where(lax.iota(jnp.int32, 16) == 0, 1.0, 0.0)

    def _orow(i, _):
        ones_buf[i, :] = onehot
        return 0
    lax.fori_loop(0, CHUNK, _orow, 0)
    plsc.subcore_barrier()

    base0 = wid * EPW

    cp0 = pltpu.async_copy(src3.at[wid], src_all, isem)
    cp1 = pltpu.async_copy(dst3.at[wid], dst_all, isem)
    cp0.wait()
    cp1.wait()

    def _compute(s):
        def _row(i, _):
            h = abufs[s][i, :] + bbufs[s][i, :] + cbufs[s][i, :]
            hbufs[s][i, :] = jnp.maximum(h, 0.2 * h)
            return 0
        lax.fori_loop(0, CHUNK, _row, 0)

    def _group(grp, _):
        t0 = grp * G
        gcps = []
        for k in range(G):
            t = t0 + k
            gcps.append((
                pltpu.async_copy(a_tab.at[src_all.at[t]], abufs[k],
                                 gsems[k]),
                pltpu.async_copy(b_tab.at[dst_all.at[t]], bbufs[k],
                                 gsems[k]),
                pltpu.async_copy(c_hbm.at[pl.ds(base0 + t * CHUNK, CHUNK)],
                                 cbufs[k], gsems[k]),
            ))
        for k in range(G):
            t = t0 + k
            for j in range(CHUNK // 16):
                dbufs[k][pl.ds(j * 16, 16)] = dst_all[t, pl.ds(j * 16, 16)]
            for cp in gcps[k]:
                cp.wait()
            _compute(k)
            if o2i:
                pltpu.sync_copy(hbufs[k], accs[0].at[dbufs[k]], add=True)
                pltpu.sync_copy(ones_buf, accs[1].at[dbufs[k]], add=True)
            else:
                pltpu.sync_copy(
                    hbufs[k], hout.at[pl.ds(base0 + t * CHUNK, CHUNK)])
                pltpu.sync_copy(ones_buf, accs[0].at[dbufs[k]], add=True)
        return 0
    lax.fori_loop(0, NCHUNK // G, _group, 0)

    plsc.subcore_barrier()

    for acc, out in zip(accs, outs):
        pltpu.sync_copy(acc.at[pl.ds(r0, RPS)], stage)
        pltpu.sync_copy(stage, out.at[cid, pl.ds(r0, RPS)])

    @pl.when(sid == NS - 1)
    def _out_tail():
        tail0 = NS * RPS
        for acc, out in zip(accs, outs):
            pltpu.sync_copy(acc.at[pl.ds(tail0, TAIL)],
                            stage.at[pl.ds(0, TAIL)])
            pltpu.sync_copy(stage.at[pl.ds(0, TAIL)],
                            out.at[cid, pl.ds(tail0, TAIL)])


def _sc_o2i_body(ao, bo, co, src3, dst3, sop, dgo,
                 src_all, dst_all,
                 a0, a1, a2, a3, a4, b0, b1, b2, b3, b4,
                 c0, c1, c2, c3, c4, h0, h1, h2, h3, h4,
                 d0, d1, d2, d3, d4, ones_buf, stage, acc_h, acc_d,
                 gsem0, gsem1, gsem2, gsem3, gsem4, isem):
    _sc_pass_impl(True, ao, bo, co, src3, dst3, (sop, dgo), None,
                  src_all, dst_all,
                  (a0, a1, a2, a3, a4), (b0, b1, b2, b3, b4),
                  (c0, c1, c2, c3, c4), (h0, h1, h2, h3, h4),
                  (d0, d1, d2, d3, d4), ones_buf, stage, (acc_h, acc_d),
                  (gsem0, gsem1, gsem2, gsem3, gsem4), isem)


def _sc_i2o_body(ai, bi, ci, src3, dst3, dgi, hout,
                 src_all, dst_all,
                 a0, a1, a2, a3, a4, b0, b1, b2, b3, b4,
                 c0, c1, c2, c3, c4, h0, h1, h2, h3, h4,
                 d0, d1, d2, d3, d4, ones_buf, stage, acc_d,
                 gsem0, gsem1, gsem2, gsem3, gsem4, isem):
    _sc_pass_impl(False, ai, bi, ci, src3, dst3, (dgi,), hout,
                  src_all, dst_all,
                  (a0, a1, a2, a3, a4), (b0, b1, b2, b3, b4),
                  (c0, c1, c2, c3, c4), (h0, h1, h2, h3, h4),
                  (d0, d1, d2, d3, d4), ones_buf, stage, (acc_d,),
                  (gsem0, gsem1, gsem2, gsem3, gsem4), isem)


def _sc_scatter_body(g_hbm, dst3, tg,
                     dst_all,
                     g0, g1, g2, g3, g4, d0, d1, d2, d3, d4,
                     stage, t_acc,
                     gsem0, gsem1, gsem2, gsem3, gsem4, isem):
    cid = lax.axis_index("c")
    sid = lax.axis_index("s")
    wid = sid * NC + cid
    gbufs = (g0, g1, g2, g3, g4)
    dbufs = (d0, d1, d2, d3, d4)
    gsems = (gsem0, gsem1, gsem2, gsem3, gsem4)

    def _zrow(i, _):
        stage[i, :] = jnp.zeros((16,), jnp.float32)
        return 0
    lax.fori_loop(0, RPS, _zrow, 0)
    r0 = sid * RPS
    pltpu.sync_copy(stage, t_acc.at[pl.ds(r0, RPS)])

    @pl.when(sid == NS - 1)
    def _zero_tail():
        pltpu.sync_copy(stage.at[pl.ds(0, TAIL)],
                        t_acc.at[pl.ds(NS * RPS, TAIL)])
    plsc.subcore_barrier()

    base0 = wid * EPW
    cp = pltpu.async_copy(dst3.at[wid], dst_all, isem)
    cp.wait()

    def _group(grp, _):
        t0 = grp * G
        gcps = []
        for k in range(G):
            t = t0 + k
            gcps.append(pltpu.async_copy(
                g_hbm.at[pl.ds(base0 + t * CHUNK, CHUNK)], gbufs[k],
                gsems[k]))
        for k in range(G):
            t = t0 + k
            for j in range(CHUNK // 16):
                dbufs[k][pl.ds(j * 16, 16)] = dst_all[t, pl.ds(j * 16, 16)]
            gcps[k].wait()
            pltpu.sync_copy(gbufs[k], t_acc.at[dbufs[k]], add=True)
        return 0
    lax.fori_loop(0, NCHUNK // G, _group, 0)

    plsc.subcore_barrier()
    pltpu.sync_copy(t_acc.at[pl.ds(r0, RPS)], stage)
    pltpu.sync_copy(stage, tg.at[cid, pl.ds(r0, RPS)])

    @pl.when(sid == NS - 1)
    def _out_tail():
        tail0 = NS * RPS
        pltpu.sync_copy(t_acc.at[pl.ds(tail0, TAIL)], stage.at[pl.ds(0, TAIL)])
        pltpu.sync_copy(stage.at[pl.ds(0, TAIL)], tg.at[cid, pl.ds(tail0, TAIL)])


_SC_MESH = plsc.VectorSubcoreMesh(core_axis_name="c", subcore_axis_name="s")
_SC_PARAMS = pltpu.CompilerParams(use_tc_tiling_on_sc=False)

_EDGE_SCRATCH = (
    [pltpu.VMEM((NCHUNK, CHUNK), jnp.int32)] * 2
    + [pltpu.VMEM((CHUNK, 16), jnp.float32)] * (4 * G)
    + [pltpu.VMEM((CHUNK,), jnp.int32)] * G
    + [pltpu.VMEM((CHUNK, 16), jnp.float32)]
    + [pltpu.VMEM((RPS, 16), jnp.float32)]
)

_sc_o2i = pl.kernel(
    _sc_o2i_body,
    out_type=(
        jax.ShapeDtypeStruct((NC, N, 16), jnp.float32),
        jax.ShapeDtypeStruct((NC, N, 16), jnp.float32),
    ),
    mesh=_SC_MESH,
    scratch_types=(
        _EDGE_SCRATCH
        + [pltpu.VMEM_SHARED((N, 16), jnp.float32)] * 2
        + [pltpu.SemaphoreType.DMA] * (G + 1)
    ),
    compiler_params=_SC_PARAMS,
)

_sc_i2o = pl.kernel(
    _sc_i2o_body,
    out_type=(
        jax.ShapeDtypeStruct((NC, N, 16), jnp.float32),
        jax.ShapeDtypeStruct((E, 16), jnp.float32),
    ),
    mesh=_SC_MESH,
    scratch_types=(
        _EDGE_SCRATCH
        + [pltpu.VMEM_SHARED((N, 16), jnp.float32)]
        + [pltpu.SemaphoreType.DMA] * (G + 1)
    ),
    compiler_params=_SC_PARAMS,
)

_sc_scatter = pl.kernel(
    _sc_scatter_body,
    out_type=jax.ShapeDtypeStruct((NC, N, 16), jnp.float32),
    mesh=_SC_MESH,
    scratch_types=(
        [pltpu.VMEM((NCHUNK, CHUNK), jnp.int32)]
        + [pltpu.VMEM((CHUNK, 16), jnp.float32)] * G
        + [pltpu.VMEM((CHUNK,), jnp.int32)] * G
        + [pltpu.VMEM((RPS, 16), jnp.float32)]
        + [pltpu.VMEM_SHARED((N, 16), jnp.float32)]
        + [pltpu.SemaphoreType.DMA] * (G + 1)
    ),
    compiler_params=_SC_PARAMS,
)


E8 = E // 8
EB8 = 5000


@jax.jit
def kernel(nf, edge_index_out, nef_out, edge_index_in, nef_in,
           w1_o2i, b1_o2i, w2_o2i, b2_o2i,
           w1_i2o, b1_i2o, w2_i2o, b2_i2o,
           w1_red, b1_red, w2_red, b2_red):
    nf = nf.astype(jnp.float32)
    src_o, dst_o = edge_index_out[0], edge_index_out[1]
    src_i, dst_i = edge_index_in[0], edge_index_in[1]

    wcat = jnp.concatenate([w1_o2i[:D], w1_o2i[D:2 * D],
                            w1_i2o[:D], w1_i2o[D:2 * D]], axis=1)
    proj = pl.pallas_call(
        _proj_body,
        out_shape=jax.ShapeDtypeStruct((N, 64), jnp.float32),
    )(nf, wcat)
    ao, bo = proj[:, 0:16], proj[:, 16:32]
    ai, bi = proj[:, 32:48], proj[:, 48:64]

    eye8 = jnp.eye(8, dtype=jnp.float32)

    def _edge_bias(nef, w16, b16):
        c8 = pl.pallas_call(
            _edge_bias_body,
            grid=(E8 // EB8,),
            in_specs=[
                pl.BlockSpec((EB8, D), lambda e: (e, 0)),
                pl.BlockSpec((D, D), lambda e: (0, 0)),
                pl.BlockSpec((1, D), lambda e: (0, 0)),
            ],
            out_specs=pl.BlockSpec((EB8, D), lambda e: (e, 0)),
            out_shape=jax.ShapeDtypeStruct((E8, D), jnp.float32),
        )(nef.reshape(E8, D), jnp.kron(eye8, w16),
          jnp.tile(b16, 8).reshape(1, D))
        return c8.reshape(E, 16)

    ci = _edge_bias(nef_in, w1_i2o[2 * D:], b1_i2o)
    co = _edge_bias(nef_out, w1_o2i[2 * D:], b1_o2i)

    dgi, h_i = _sc_i2o(
        ai, bi, ci,
        src_i.reshape(NW, NCHUNK, CHUNK), dst_i.reshape(NW, NCHUNK, CHUNK))
    sop, dgo = _sc_o2i(
        ao, bo, co,
        src_o.reshape(NW, NCHUNK, CHUNK), dst_o.reshape(NW, NCHUNK, CHUNK))

    wk8 = jnp.kron(eye8, w2_i2o[:, 0:1] * jnp.ones((1, 16), jnp.float32))
    g8 = pl.pallas_call(
        _gate_body,
        grid=(E8 // EB8,),
        in_specs=[
            pl.BlockSpec((EB8, D), lambda e: (e, 0)),
            pl.BlockSpec((D, D), lambda e: (0, 0)),
            pl.BlockSpec((1, 1), lambda e: (0, 0)),
            pl.BlockSpec((D, D), lambda e: (0, 0)),
            pl.BlockSpec((1, D), lambda e: (0, 0)),
        ],
        out_specs=pl.BlockSpec((EB8, D), lambda e: (e, 0)),
        out_shape=jax.ShapeDtypeStruct((E8, D), jnp.float32),
    )(h_i.reshape(E8, D), wk8, b2_i2o[0].reshape(1, 1),
      jnp.kron(eye8, w2_i2o[:, 1:]), jnp.tile(b2_i2o[1:], 8).reshape(1, D))
    g = g8.reshape(E, 16)

    tg = _sc_scatter(g, dst_i.reshape(NW, NCHUNK, CHUNK))

    nb = 2000
    out = pl.pallas_call(
        _final_body,
        grid=(N // nb,),
        in_specs=[
            pl.BlockSpec((NC, nb, 16), lambda n: (0, n, 0)),
            pl.BlockSpec((NC, nb, 16), lambda n: (0, n, 0)),
            pl.BlockSpec((NC, nb, 16), lambda n: (0, n, 0)),
            pl.BlockSpec((NC, nb, 16), lambda n: (0, n, 0)),
            pl.BlockSpec((16, D), lambda n: (0, 0)),
            pl.BlockSpec((1, D), lambda n: (0, 0)),
            pl.BlockSpec((D, HIN), lambda n: (0, 0)),
            pl.BlockSpec((16, HIN), lambda n: (0, 0)),
            pl.BlockSpec((1, HIN), lambda n: (0, 0)),
            pl.BlockSpec((HIN, D), lambda n: (0, 0)),
            pl.BlockSpec((1, D), lambda n: (0, 0)),
        ],
        out_specs=pl.BlockSpec((nb, D), lambda n: (n, 0)),
        out_shape=jax.ShapeDtypeStruct((N, D), jnp.float32),
    )(sop, dgo, dgi, tg,
      w2_o2i, b2_o2i.reshape(1, D),
      w1_red[:D], w1_red[D:], b1_red.reshape(1, HIN),
      w2_red, b2_red.reshape(1, D))
    return out


# speedup vs baseline: 9.9337x; 1.0739x over previous
import functools
import jax
import jax.numpy as jnp
from jax import lax
from jax.experimental import pallas as pl
from jax.experimental.pallas import tpu as pltpu
from jax.experimental.pallas import tpu_sc as plsc

N = 10000
E = 320000
D = 128
HIN = 16

NC = 2
NS = 16
NW = NC * NS
EPW = E // NW
CHUNK = 80
NCHUNK = EPW // CHUNK
G = 5
RPS = 624
TAIL = N - NS * RPS


def _proj_body(nf_ref, w_ref, ao_ref, bo_ref, ai_ref, bi_ref):
    p = jnp.dot(nf_ref[...], w_ref[...], preferred_element_type=jnp.float32)
    ao_ref[...] = p[:, 0:16]
    bo_ref[...] = p[:, 16:32]
    ai_ref[...] = p[:, 32:48]
    bi_ref[...] = p[:, 48:64]


def _wprep_body(wco_ref, wci_ref, wg_ref, wk_ref,
                wco8_ref, wci8_ref, wg8_ref, wk8_ref):
    r = lax.broadcasted_iota(jnp.int32, (D, D), 0)
    c = lax.broadcasted_iota(jnp.int32, (D, D), 1)
    mask = (r // 16) == (c // 16)

    def bd(w16):
        return jnp.where(mask, jnp.tile(w16, (8, 8)), 0.0)

    wco8_ref[...] = bd(wco_ref[...])
    wci8_ref[...] = bd(wci_ref[...])
    wg8_ref[...] = bd(wg_ref[...])
    wk8_ref[...] = bd(jnp.tile(wk_ref[...], (1, 16)))


def _edge_bias_body(nef_ref, w_ref, b_ref, c_ref):
    c_ref[...] = jnp.dot(nef_ref[...], w_ref[...],
                         preferred_element_type=jnp.float32) + b_ref[...]


def _gate_body(h_ref, wk_ref, bk_ref, wg_ref, bg_ref, g_ref):
    h = h_ref[...]
    m0 = jnp.dot(h, wk_ref[...], preferred_element_type=jnp.float32)
    k = jax.nn.sigmoid(m0 + bk_ref[...])
    g_ref[...] = (jnp.dot(h, wg_ref[...],
                          preferred_element_type=jnp.float32)
                  + bg_ref[...]) * k


def _final_body(sop_ref, dgo_ref, dgi_ref, tg_ref,
                w2o_ref, b2o_ref, w1a_ref, w1b_ref, b1r_ref,
                w2r_ref, b2r_ref, out_ref):
    hsum = sop_ref[0] + sop_ref[1]
    dego = dgo_ref[0][:, 0:1] + dgo_ref[1][:, 0:1]
    degi = dgi_ref[0][:, 0:1] + dgi_ref[1][:, 0:1]
    new_nf = (jnp.dot(hsum, w2o_ref[...],
                      preferred_element_type=jnp.float32)
              + dego * b2o_ref[...])
    t = tg_ref[0] + tg_ref[1]
    lane = lax.broadcasted_iota(jnp.int32, t.shape, 1)
    scale = jnp.where(lane < 8, 1.0, 1.0 / jnp.maximum(degi, 1.0))
    ts = t * scale
    hr = (jnp.dot(new_nf, w1a_ref[...], preferred_element_type=jnp.float32)
          + jnp.dot(ts, w1b_ref[...], preferred_element_type=jnp.float32)
          + b1r_ref[...])
    hr = jnp.maximum(hr, 0.2 * hr)
    red = jnp.dot(hr, w2r_ref[...],
                  preferred_element_type=jnp.float32) + b2r_ref[...]
    out_ref[...] = jnp.where(degi > 0, red, new_nf)


def _sc_pass_impl(o2i, a_tab, b_tab, c_hbm, src3, dst3, outs, hout,
                  src_all, dst_all, abufs, bbufs, cbufs, hbufs, dbufs,
                  ones_buf, stage, accs, gsems, isem):
    cid = lax.axis_index("c")
    sid = lax.axis_index("s")
    wid = sid * NC + cid

    def _zrow(i, _):
        stage[i, :] = jnp.zeros((16,), jnp.float32)
        return 0
    lax.fori_loop(0, RPS, _zrow, 0)
    r0 = sid * RPS
    for acc in accs:
        pltpu.sync_copy(stage, acc.at[pl.ds(r0, RPS)])

    @pl.when(sid == NS - 1)
    def _zero_tail():
        for acc in accs:
            pltpu.sync_copy(stage.at[pl.ds(0, TAIL)],
                            acc.at[pl.ds(NS * RPS, TAIL)])

    onehot = jnp.where(lax.iota(jnp.int32, 16) == 0, 1.0, 0.0)

    def _orow(i, _):
        ones_buf[i, :] = onehot
        return 0
    lax.fori_loop(0, CHUNK, _orow, 0)
    plsc.subcore_barrier()

    base0 = wid * EPW

    cp0 = pltpu.async_copy(src3.at[wid], src_all, isem)
    cp1 = pltpu.async_copy(dst3.at[wid], dst_all, isem)
    cp0.wait()
    cp1.wait()

    def _compute(s):
        def _row8(i8, _):
            for u in range(8):
                i = i8 * 8 + u
                h = abufs[s][i, :] + bbufs[s][i, :] + cbufs[s][i, :]
                hbufs[s][i, :] = jnp.maximum(h, 0.2 * h)
            return 0
        lax.fori_loop(0, CHUNK // 8, _row8, 0)

    def _group(grp, _):
        t0 = grp * G
        gcps = []
        for k in range(G):
            t = t0 + k
            gcps.append((
                pltpu.async_copy(a_tab.at[src_all.at[t]], abufs[k],
                                 gsems[k]),
                pltpu.async_copy(b_tab.at[dst_all.at[t]], bbufs[k],
                                 gsems[k]),
                pltpu.async_copy(c_hbm.at[pl.ds(base0 + t * CHUNK, CHUNK)],
                                 cbufs[k], gsems[k]),
            ))
        for k in range(G):
            t = t0 + k
            for j in range(CHUNK // 16):
                dbufs[k][pl.ds(j * 16, 16)] = dst_all[t, pl.ds(j * 16, 16)]
            for cp in gcps[k]:
                cp.wait()
            _compute(k)
            if o2i:
                pltpu.sync_copy(hbufs[k], accs[0].at[dbufs[k]], add=True)
                pltpu.sync_copy(ones_buf, accs[1].at[dbufs[k]], add=True)
            else:
                pltpu.sync_copy(
                    hbufs[k], hout.at[pl.ds(base0 + t * CHUNK, CHUNK)])
                pltpu.sync_copy(ones_buf, accs[0].at[dbufs[k]], add=True)
        return 0
    lax.fori_loop(0, NCHUNK // G, _group, 0)

    plsc.subcore_barrier()

    for acc, out in zip(accs, outs):
        pltpu.sync_copy(acc.at[pl.ds(r0, RPS)], stage)
        pltpu.sync_copy(stage, out.at[cid, pl.ds(r0, RPS)])

    @pl.when(sid == NS - 1)
    def _out_tail():
        tail0 = NS * RPS
        for acc, out in zip(accs, outs):
            pltpu.sync_copy(acc.at[pl.ds(tail0, TAIL)],
                            stage.at[pl.ds(0, TAIL)])
            pltpu.sync_copy(stage.at[pl.ds(0, TAIL)],
                            out.at[cid, pl.ds(tail0, TAIL)])


def _sc_o2i_body(ao, bo, co, src3, dst3, sop, dgo,
                 src_all, dst_all,
                 a0, a1, a2, a3, a4, b0, b1, b2, b3, b4,
                 c0, c1, c2, c3, c4, h0, h1, h2, h3, h4,
                 d0, d1, d2, d3, d4, ones_buf, stage, acc_h, acc_d,
                 gsem0, gsem1, gsem2, gsem3, gsem4, isem):
    _sc_pass_impl(True, ao, bo, co, src3, dst3, (sop, dgo), None,
                  src_all, dst_all,
                  (a0, a1, a2, a3, a4), (b0, b1, b2, b3, b4),
                  (c0, c1, c2, c3, c4), (h0, h1, h2, h3, h4),
                  (d0, d1, d2, d3, d4), ones_buf, stage, (acc_h, acc_d),
                  (gsem0, gsem1, gsem2, gsem3, gsem4), isem)


def _sc_i2o_body(ai, bi, ci, src3, dst3, dgi, hout,
                 src_all, dst_all,
                 a0, a1, a2, a3, a4, b0, b1, b2, b3, b4,
                 c0, c1, c2, c3, c4, h0, h1, h2, h3, h4,
                 d0, d1, d2, d3, d4, ones_buf, stage, acc_d,
                 gsem0, gsem1, gsem2, gsem3, gsem4, isem):
    _sc_pass_impl(False, ai, bi, ci, src3, dst3, (dgi,), hout,
                  src_all, dst_all,
                  (a0, a1, a2, a3, a4), (b0, b1, b2, b3, b4),
                  (c0, c1, c2, c3, c4), (h0, h1, h2, h3, h4),
                  (d0, d1, d2, d3, d4), ones_buf, stage, (acc_d,),
                  (gsem0, gsem1, gsem2, gsem3, gsem4), isem)


def _sc_scatter_body(g_hbm, dst3, tg,
                     dst_all,
                     g0, g1, g2, g3, g4, d0, d1, d2, d3, d4,
                     stage, t_acc,
                     gsem0, gsem1, gsem2, gsem3, gsem4, isem):
    cid = lax.axis_index("c")
    sid = lax.axis_index("s")
    wid = sid * NC + cid
    gbufs = (g0, g1, g2, g3, g4)
    dbufs = (d0, d1, d2, d3, d4)
    gsems = (gsem0, gsem1, gsem2, gsem3, gsem4)

    def _zrow(i, _):
        stage[i, :] = jnp.zeros((16,), jnp.float32)
        return 0
    lax.fori_loop(0, RPS, _zrow, 0)
    r0 = sid * RPS
    pltpu.sync_copy(stage, t_acc.at[pl.ds(r0, RPS)])

    @pl.when(sid == NS - 1)
    def _zero_tail():
        pltpu.sync_copy(stage.at[pl.ds(0, TAIL)],
                        t_acc.at[pl.ds(NS * RPS, TAIL)])
    plsc.subcore_barrier()

    base0 = wid * EPW
    cp = pltpu.async_copy(dst3.at[wid], dst_all, isem)
    cp.wait()

    def _group(grp, _):
        t0 = grp * G
        gcps = []
        for k in range(G):
            t = t0 + k
            gcps.append(pltpu.async_copy(
                g_hbm.at[pl.ds(base0 + t * CHUNK, CHUNK)], gbufs[k],
                gsems[k]))
        for k in range(G):
            t = t0 + k
            for j in range(CHUNK // 16):
                dbufs[k][pl.ds(j * 16, 16)] = dst_all[t, pl.ds(j * 16, 16)]
            gcps[k].wait()
            pltpu.sync_copy(gbufs[k], t_acc.at[dbufs[k]], add=True)
        return 0
    lax.fori_loop(0, NCHUNK // G, _group, 0)

    plsc.subcore_barrier()
    pltpu.sync_copy(t_acc.at[pl.ds(r0, RPS)], stage)
    pltpu.sync_copy(stage, tg.at[cid, pl.ds(r0, RPS)])

    @pl.when(sid == NS - 1)
    def _out_tail():
        tail0 = NS * RPS
        pltpu.sync_copy(t_acc.at[pl.ds(tail0, TAIL)], stage.at[pl.ds(0, TAIL)])
        pltpu.sync_copy(stage.at[pl.ds(0, TAIL)], tg.at[cid, pl.ds(tail0, TAIL)])


_SC_MESH = plsc.VectorSubcoreMesh(core_axis_name="c", subcore_axis_name="s")
_SC_PARAMS = pltpu.CompilerParams(use_tc_tiling_on_sc=False)

_EDGE_SCRATCH = (
    [pltpu.VMEM((NCHUNK, CHUNK), jnp.int32)] * 2
    + [pltpu.VMEM((CHUNK, 16), jnp.float32)] * (4 * G)
    + [pltpu.VMEM((CHUNK,), jnp.int32)] * G
    + [pltpu.VMEM((CHUNK, 16), jnp.float32)]
    + [pltpu.VMEM((RPS, 16), jnp.float32)]
)

_sc_o2i = pl.kernel(
    _sc_o2i_body,
    out_type=(
        jax.ShapeDtypeStruct((NC, N, 16), jnp.float32),
        jax.ShapeDtypeStruct((NC, N, 16), jnp.float32),
    ),
    mesh=_SC_MESH,
    scratch_types=(
        _EDGE_SCRATCH
        + [pltpu.VMEM_SHARED((N, 16), jnp.float32)] * 2
        + [pltpu.SemaphoreType.DMA] * (G + 1)
    ),
    compiler_params=_SC_PARAMS,
)

_sc_i2o = pl.kernel(
    _sc_i2o_body,
    out_type=(
        jax.ShapeDtypeStruct((NC, N, 16), jnp.float32),
        jax.ShapeDtypeStruct((E, 16), jnp.float32),
    ),
    mesh=_SC_MESH,
    scratch_types=(
        _EDGE_SCRATCH
        + [pltpu.VMEM_SHARED((N, 16), jnp.float32)]
        + [pltpu.SemaphoreType.DMA] * (G + 1)
    ),
    compiler_params=_SC_PARAMS,
)

_sc_scatter = pl.kernel(
    _sc_scatter_body,
    out_type=jax.ShapeDtypeStruct((NC, N, 16), jnp.float32),
    mesh=_SC_MESH,
    scratch_types=(
        [pltpu.VMEM((NCHUNK, CHUNK), jnp.int32)]
        + [pltpu.VMEM((CHUNK, 16), jnp.float32)] * G
        + [pltpu.VMEM((CHUNK,), jnp.int32)] * G
        + [pltpu.VMEM((RPS, 16), jnp.float32)]
        + [pltpu.VMEM_SHARED((N, 16), jnp.float32)]
        + [pltpu.SemaphoreType.DMA] * (G + 1)
    ),
    compiler_params=_SC_PARAMS,
)


E8 = E // 8
EB8 = 5000


@jax.jit
def kernel(nf, edge_index_out, nef_out, edge_index_in, nef_in,
           w1_o2i, b1_o2i, w2_o2i, b2_o2i,
           w1_i2o, b1_i2o, w2_i2o, b2_i2o,
           w1_red, b1_red, w2_red, b2_red):
    nf = nf.astype(jnp.float32)
    src_o, dst_o = edge_index_out[0], edge_index_out[1]
    src_i, dst_i = edge_index_in[0], edge_index_in[1]

    wcat = jnp.concatenate([w1_o2i[:D], w1_o2i[D:2 * D],
                            w1_i2o[:D], w1_i2o[D:2 * D]], axis=1)
    ao, bo, ai, bi = pl.pallas_call(
        _proj_body,
        out_shape=[jax.ShapeDtypeStruct((N, 16), jnp.float32)] * 4,
    )(nf, wcat)

    wco8, wci8, wg8, wk8 = pl.pallas_call(
        _wprep_body,
        out_shape=[jax.ShapeDtypeStruct((D, D), jnp.float32)] * 4,
    )(w1_o2i[2 * D:], w1_i2o[2 * D:], w2_i2o[:, 1:], w2_i2o[:, 0:1])

    def _edge_bias(nef, w8, b16):
        c8 = pl.pallas_call(
            _edge_bias_body,
            grid=(E8 // EB8,),
            in_specs=[
                pl.BlockSpec((EB8, D), lambda e: (e, 0)),
                pl.BlockSpec((D, D), lambda e: (0, 0)),
                pl.BlockSpec((1, D), lambda e: (0, 0)),
            ],
            out_specs=pl.BlockSpec((EB8, D), lambda e: (e, 0)),
            out_shape=jax.ShapeDtypeStruct((E8, D), jnp.float32),
        )(nef.reshape(E8, D), w8, jnp.tile(b16, 8).reshape(1, D))
        return c8.reshape(E, 16)

    ci = _edge_bias(nef_in, wci8, b1_i2o)
    co = _edge_bias(nef_out, wco8, b1_o2i)

    dgi, h_i = _sc_i2o(
        ai, bi, ci,
        src_i.reshape(NW, NCHUNK, CHUNK), dst_i.reshape(NW, NCHUNK, CHUNK))
    sop, dgo = _sc_o2i(
        ao, bo, co,
        src_o.reshape(NW, NCHUNK, CHUNK), dst_o.reshape(NW, NCHUNK, CHUNK))

    g8 = pl.pallas_call(
        _gate_body,
        grid=(E8 // EB8,),
        in_specs=[
            pl.BlockSpec((EB8, D), lambda e: (e, 0)),
            pl.BlockSpec((D, D), lambda e: (0, 0)),
            pl.BlockSpec((1, 1), lambda e: (0, 0)),
            pl.BlockSpec((D, D), lambda e: (0, 0)),
            pl.BlockSpec((1, D), lambda e: (0, 0)),
        ],
        out_specs=pl.BlockSpec((EB8, D), lambda e: (e, 0)),
        out_shape=jax.ShapeDtypeStruct((E8, D), jnp.float32),
    )(h_i.reshape(E8, D), wk8, b2_i2o[0].reshape(1, 1),
      wg8, jnp.tile(b2_i2o[1:], 8).reshape(1, D))
    g = g8.reshape(E, 16)

    tg = _sc_scatter(g, dst_i.reshape(NW, NCHUNK, CHUNK))

    nb = 2000
    out = pl.pallas_call(
        _final_body,
        grid=(N // nb,),
        in_specs=[
            pl.BlockSpec((NC, nb, 16), lambda n: (0, n, 0)),
            pl.BlockSpec((NC, nb, 16), lambda n: (0, n, 0)),
            pl.BlockSpec((NC, nb, 16), lambda n: (0, n, 0)),
            pl.BlockSpec((NC, nb, 16), lambda n: (0, n, 0)),
            pl.BlockSpec((16, D), lambda n: (0, 0)),
            pl.BlockSpec((1, D), lambda n: (0, 0)),
            pl.BlockSpec((D, HIN), lambda n: (0, 0)),
            pl.BlockSpec((16, HIN), lambda n: (0, 0)),
            pl.BlockSpec((1, HIN), lambda n: (0, 0)),
            pl.BlockSpec((HIN, D), lambda n: (0, 0)),
            pl.BlockSpec((1, D), lambda n: (0, 0)),
        ],
        out_specs=pl.BlockSpec((nb, D), lambda n: (n, 0)),
        out_shape=jax.ShapeDtypeStruct((N, D), jnp.float32),
    )(sop, dgo, dgi, tg,
      w2_o2i, b2_o2i.reshape(1, D),
      w1_red[:D], w1_red[D:], b1_red.reshape(1, HIN),
      w2_red, b2_red.reshape(1, D))
    return out


# speedup vs baseline: 9.9703x; 1.0037x over previous
import functools
import jax
import jax.numpy as jnp
from jax import lax
from jax.experimental import pallas as pl
from jax.experimental.pallas import tpu as pltpu
from jax.experimental.pallas import tpu_sc as plsc

N = 10000
E = 320000
D = 128
HIN = 16

NC = 2
NS = 16
NW = NC * NS
EPW = E // NW
CHUNK = 80
NCHUNK = EPW // CHUNK
G = 5
RPS = 624
TAIL = N - NS * RPS


def _proj_body(nf_ref, wo_ref, wi_ref, ao_ref, bo_ref, ai_ref, bi_ref):
    nf = nf_ref[...]
    ao_ref[...] = jnp.dot(nf, wo_ref[0:D, :],
                          preferred_element_type=jnp.float32)
    bo_ref[...] = jnp.dot(nf, wo_ref[D:2 * D, :],
                          preferred_element_type=jnp.float32)
    ai_ref[...] = jnp.dot(nf, wi_ref[0:D, :],
                          preferred_element_type=jnp.float32)
    bi_ref[...] = jnp.dot(nf, wi_ref[D:2 * D, :],
                          preferred_element_type=jnp.float32)


def _wprep_body(wo_ref, wi_ref, w2i_ref,
                wco8_ref, wci8_ref, wg8_ref, wk8_ref):
    r = lax.broadcasted_iota(jnp.int32, (D, D), 0)
    c = lax.broadcasted_iota(jnp.int32, (D, D), 1)
    mask = (r // 16) == (c // 16)

    def bd(w16):
        return jnp.where(mask, jnp.tile(w16, (8, 8)), 0.0)

    wco8_ref[...] = bd(wo_ref[2 * D:2 * D + 16, :])
    wci8_ref[...] = bd(wi_ref[2 * D:2 * D + 16, :])
    wg8_ref[...] = bd(w2i_ref[:, 1:17])
    wk8_ref[...] = bd(jnp.tile(w2i_ref[:, 0:1], (1, 16)))


def _edge_bias_body(nef_ref, w_ref, b_ref, c_ref):
    c_ref[...] = jnp.dot(nef_ref[...], w_ref[...],
                         preferred_element_type=jnp.float32) + b_ref[...]


def _gate_body(h_ref, wk_ref, bk_ref, wg_ref, bg_ref, g_ref):
    h = h_ref[...]
    m0 = jnp.dot(h, wk_ref[...], preferred_element_type=jnp.float32)
    k = jax.nn.sigmoid(m0 + bk_ref[...])
    g_ref[...] = (jnp.dot(h, wg_ref[...],
                          preferred_element_type=jnp.float32)
                  + bg_ref[...]) * k


def _final_body(sop_ref, dgo_ref, dgi_ref, tg_ref,
                w2o_ref, b2o_ref, w1r_ref, b1r_ref,
                w2r_ref, b2r_ref, out_ref):
    w1a_ref = w1r_ref.at[0:D, :]
    w1b_ref = w1r_ref.at[D:D + 16, :]
    hsum = sop_ref[0] + sop_ref[1]
    dego = dgo_ref[0][:, 0:1] + dgo_ref[1][:, 0:1]
    degi = dgi_ref[0][:, 0:1] + dgi_ref[1][:, 0:1]
    new_nf = (jnp.dot(hsum, w2o_ref[...],
                      preferred_element_type=jnp.float32)
              + dego * b2o_ref[...])
    t = tg_ref[0] + tg_ref[1]
    lane = lax.broadcasted_iota(jnp.int32, t.shape, 1)
    scale = jnp.where(lane < 8, 1.0, 1.0 / jnp.maximum(degi, 1.0))
    ts = t * scale
    hr = (jnp.dot(new_nf, w1a_ref[...], preferred_element_type=jnp.float32)
          + jnp.dot(ts, w1b_ref[...], preferred_element_type=jnp.float32)
          + b1r_ref[...])
    hr = jnp.maximum(hr, 0.2 * hr)
    red = jnp.dot(hr, w2r_ref[...],
                  preferred_element_type=jnp.float32) + b2r_ref[...]
    out_ref[...] = jnp.where(degi > 0, red, new_nf)


def _sc_pass_impl(o2i, a_tab, b_tab, c_hbm, src3, dst3, outs, hout,
                  src_all, dst_all, abufs, bbufs, cbufs, hbufs, dbufs,
                  ones_buf, stage, accs, gsems, isem):
    cid = lax.axis_index("c")
    sid = lax.axis_index("s")
    wid = sid * NC + cid

    def _zrow(i, _):
        stage[i, :] = jnp.zeros((16,), jnp.float32)
        return 0
    lax.fori_loop(0, RPS, _zrow, 0)
    r0 = sid * RPS
    for acc in accs:
        pltpu.sync_copy(stage, acc.at[pl.ds(r0, RPS)])

    @pl.when(sid == NS - 1)
    def _zero_tail():
        for acc in accs:
            pltpu.sync_copy(stage.at[pl.ds(0, TAIL)],
                            acc.at[pl.ds(NS * RPS, TAIL)])

    onehot = jnp.where(lax.iota(jnp.int32, 16) == 0, 1.0, 0.0)

    def _orow(i, _):
        ones_buf[i, :] = onehot
        return 0
    lax.fori_loop(0, CHUNK, _orow, 0)
    plsc.subcore_barrier()

    base0 = wid * EPW

    cp0 = pltpu.async_copy(src3.at[wid], src_all, isem)
    cp1 = pltpu.async_copy(dst3.at[wid], dst_all, isem)
    cp0.wait()
    cp1.wait()

    def _compute(s):
        def _row8(i8, _):
            for u in range(8):
                i = i8 * 8 + u
                h = abufs[s][i, :] + bbufs[s][i, :] + cbufs[s][i, :]
                hbufs[s][i, :] = jnp.maximum(h, 0.2 * h)
            return 0
        lax.fori_loop(0, CHUNK // 8, _row8, 0)

    def _group(grp, _):
        t0 = grp * G
        gcps = []
        for k in range(G):
            t = t0 + k
            gcps.append((
                pltpu.async_copy(a_tab.at[src_all.at[t]], abufs[k],
                                 gsems[k]),
                pltpu.async_copy(b_tab.at[dst_all.at[t]], bbufs[k],
                                 gsems[k]),
                pltpu.async_copy(c_hbm.at[pl.ds(base0 + t * CHUNK, CHUNK)],
                                 cbufs[k], gsems[k]),
            ))
        for k in range(G):
            t = t0 + k
            for j in range(CHUNK // 16):
                dbufs[k][pl.ds(j * 16, 16)] = dst_all[t, pl.ds(j * 16, 16)]
            for cp in gcps[k]:
                cp.wait()
            _compute(k)
            if o2i:
                pltpu.sync_copy(hbufs[k], accs[0].at[dbufs[k]], add=True)
                pltpu.sync_copy(ones_buf, accs[1].at[dbufs[k]], add=True)
            else:
                pltpu.sync_copy(
                    hbufs[k], hout.at[pl.ds(base0 + t * CHUNK, CHUNK)])
                pltpu.sync_copy(ones_buf, accs[0].at[dbufs[k]], add=True)
        return 0
    lax.fori_loop(0, NCHUNK // G, _group, 0)

    plsc.subcore_barrier()

    for acc, out in zip(accs, outs):
        pltpu.sync_copy(acc.at[pl.ds(r0, RPS)], stage)
        pltpu.sync_copy(stage, out.at[cid, pl.ds(r0, RPS)])

    @pl.when(sid == NS - 1)
    def _out_tail():
        tail0 = NS * RPS
        for acc, out in zip(accs, outs):
            pltpu.sync_copy(acc.at[pl.ds(tail0, TAIL)],
                            stage.at[pl.ds(0, TAIL)])
            pltpu.sync_copy(stage.at[pl.ds(0, TAIL)],
                            out.at[cid, pl.ds(tail0, TAIL)])


def _sc_o2i_body(ao, bo, co, src3, dst3, sop, dgo,
                 src_all, dst_all,
                 a0, a1, a2, a3, a4, b0, b1, b2, b3, b4,
                 c0, c1, c2, c3, c4, h0, h1, h2, h3, h4,
                 d0, d1, d2, d3, d4, ones_buf, stage, acc_h, acc_d,
                 gsem0, gsem1, gsem2, gsem3, gsem4, isem):
    _sc_pass_impl(True, ao, bo, co, src3, dst3, (sop, dgo), None,
                  src_all, dst_all,
                  (a0, a1, a2, a3, a4), (b0, b1, b2, b3, b4),
                  (c0, c1, c2, c3, c4), (h0, h1, h2, h3, h4),
                  (d0, d1, d2, d3, d4), ones_buf, stage, (acc_h, acc_d),
                  (gsem0, gsem1, gsem2, gsem3, gsem4), isem)


def _sc_i2o_body(ai, bi, ci, src3, dst3, dgi, hout,
                 src_all, dst_all,
                 a0, a1, a2, a3, a4, b0, b1, b2, b3, b4,
                 c0, c1, c2, c3, c4, h0, h1, h2, h3, h4,
                 d0, d1, d2, d3, d4, ones_buf, stage, acc_d,
                 gsem0, gsem1, gsem2, gsem3, gsem4, isem):
    _sc_pass_impl(False, ai, bi, ci, src3, dst3, (dgi,), hout,
                  src_all, dst_all,
                  (a0, a1, a2, a3, a4), (b0, b1, b2, b3, b4),
                  (c0, c1, c2, c3, c4), (h0, h1, h2, h3, h4),
                  (d0, d1, d2, d3, d4), ones_buf, stage, (acc_d,),
                  (gsem0, gsem1, gsem2, gsem3, gsem4), isem)


def _sc_scatter_body(g_hbm, dst3, tg,
                     dst_all,
                     g0, g1, g2, g3, g4, d0, d1, d2, d3, d4,
                     stage, t_acc,
                     gsem0, gsem1, gsem2, gsem3, gsem4, isem):
    cid = lax.axis_index("c")
    sid = lax.axis_index("s")
    wid = sid * NC + cid
    gbufs = (g0, g1, g2, g3, g4)
    dbufs = (d0, d1, d2, d3, d4)
    gsems = (gsem0, gsem1, gsem2, gsem3, gsem4)

    def _zrow(i, _):
        stage[i, :] = jnp.zeros((16,), jnp.float32)
        return 0
    lax.fori_loop(0, RPS, _zrow, 0)
    r0 = sid * RPS
    pltpu.sync_copy(stage, t_acc.at[pl.ds(r0, RPS)])

    @pl.when(sid == NS - 1)
    def _zero_tail():
        pltpu.sync_copy(stage.at[pl.ds(0, TAIL)],
                        t_acc.at[pl.ds(NS * RPS, TAIL)])
    plsc.subcore_barrier()

    base0 = wid * EPW
    cp = pltpu.async_copy(dst3.at[wid], dst_all, isem)
    cp.wait()

    def _group(grp, _):
        t0 = grp * G
        gcps = []
        for k in range(G):
            t = t0 + k
            gcps.append(pltpu.async_copy(
                g_hbm.at[pl.ds(base0 + t * CHUNK, CHUNK)], gbufs[k],
                gsems[k]))
        for k in range(G):
            t = t0 + k
            for j in range(CHUNK // 16):
                dbufs[k][pl.ds(j * 16, 16)] = dst_all[t, pl.ds(j * 16, 16)]
            gcps[k].wait()
            pltpu.sync_copy(gbufs[k], t_acc.at[dbufs[k]], add=True)
        return 0
    lax.fori_loop(0, NCHUNK // G, _group, 0)

    plsc.subcore_barrier()
    pltpu.sync_copy(t_acc.at[pl.ds(r0, RPS)], stage)
    pltpu.sync_copy(stage, tg.at[cid, pl.ds(r0, RPS)])

    @pl.when(sid == NS - 1)
    def _out_tail():
        tail0 = NS * RPS
        pltpu.sync_copy(t_acc.at[pl.ds(tail0, TAIL)], stage.at[pl.ds(0, TAIL)])
        pltpu.sync_copy(stage.at[pl.ds(0, TAIL)], tg.at[cid, pl.ds(tail0, TAIL)])


_SC_MESH = plsc.VectorSubcoreMesh(core_axis_name="c", subcore_axis_name="s")
_SC_PARAMS = pltpu.CompilerParams(use_tc_tiling_on_sc=False)

_EDGE_SCRATCH = (
    [pltpu.VMEM((NCHUNK, CHUNK), jnp.int32)] * 2
    + [pltpu.VMEM((CHUNK, 16), jnp.float32)] * (4 * G)
    + [pltpu.VMEM((CHUNK,), jnp.int32)] * G
    + [pltpu.VMEM((CHUNK, 16), jnp.float32)]
    + [pltpu.VMEM((RPS, 16), jnp.float32)]
)

_sc_o2i = pl.kernel(
    _sc_o2i_body,
    out_type=(
        jax.ShapeDtypeStruct((NC, N, 16), jnp.float32),
        jax.ShapeDtypeStruct((NC, N, 16), jnp.float32),
    ),
    mesh=_SC_MESH,
    scratch_types=(
        _EDGE_SCRATCH
        + [pltpu.VMEM_SHARED((N, 16), jnp.float32)] * 2
        + [pltpu.SemaphoreType.DMA] * (G + 1)
    ),
    compiler_params=_SC_PARAMS,
)

_sc_i2o = pl.kernel(
    _sc_i2o_body,
    out_type=(
        jax.ShapeDtypeStruct((NC, N, 16), jnp.float32),
        jax.ShapeDtypeStruct((E, 16), jnp.float32),
    ),
    mesh=_SC_MESH,
    scratch_types=(
        _EDGE_SCRATCH
        + [pltpu.VMEM_SHARED((N, 16), jnp.float32)]
        + [pltpu.SemaphoreType.DMA] * (G + 1)
    ),
    compiler_params=_SC_PARAMS,
)

_sc_scatter = pl.kernel(
    _sc_scatter_body,
    out_type=jax.ShapeDtypeStruct((NC, N, 16), jnp.float32),
    mesh=_SC_MESH,
    scratch_types=(
        [pltpu.VMEM((NCHUNK, CHUNK), jnp.int32)]
        + [pltpu.VMEM((CHUNK, 16), jnp.float32)] * G
        + [pltpu.VMEM((CHUNK,), jnp.int32)] * G
        + [pltpu.VMEM((RPS, 16), jnp.float32)]
        + [pltpu.VMEM_SHARED((N, 16), jnp.float32)]
        + [pltpu.SemaphoreType.DMA] * (G + 1)
    ),
    compiler_params=_SC_PARAMS,
)


E8 = E // 8
EB8 = 5000


@jax.jit
def kernel(nf, edge_index_out, nef_out, edge_index_in, nef_in,
           w1_o2i, b1_o2i, w2_o2i, b2_o2i,
           w1_i2o, b1_i2o, w2_i2o, b2_i2o,
           w1_red, b1_red, w2_red, b2_red):
    nf = nf.astype(jnp.float32)
    src_o, dst_o = edge_index_out[0], edge_index_out[1]
    src_i, dst_i = edge_index_in[0], edge_index_in[1]

    ao, bo, ai, bi = pl.pallas_call(
        _proj_body,
        out_shape=[jax.ShapeDtypeStruct((N, 16), jnp.float32)] * 4,
    )(nf, w1_o2i, w1_i2o)

    wco8, wci8, wg8, wk8 = pl.pallas_call(
        _wprep_body,
        out_shape=[jax.ShapeDtypeStruct((D, D), jnp.float32)] * 4,
    )(w1_o2i, w1_i2o, w2_i2o)

    def _edge_bias(nef, w8, b16):
        c8 = pl.pallas_call(
            _edge_bias_body,
            grid=(E8 // EB8,),
            in_specs=[
                pl.BlockSpec((EB8, D), lambda e: (e, 0)),
                pl.BlockSpec((D, D), lambda e: (0, 0)),
                pl.BlockSpec((1, D), lambda e: (0, 0)),
            ],
            out_specs=pl.BlockSpec((EB8, D), lambda e: (e, 0)),
            out_shape=jax.ShapeDtypeStruct((E8, D), jnp.float32),
        )(nef.reshape(E8, D), w8, jnp.tile(b16, 8).reshape(1, D))
        return c8.reshape(E, 16)

    ci = _edge_bias(nef_in, wci8, b1_i2o)
    co = _edge_bias(nef_out, wco8, b1_o2i)

    dgi, h_i = _sc_i2o(
        ai, bi, ci,
        src_i.reshape(NW, NCHUNK, CHUNK), dst_i.reshape(NW, NCHUNK, CHUNK))
    sop, dgo = _sc_o2i(
        ao, bo, co,
        src_o.reshape(NW, NCHUNK, CHUNK), dst_o.reshape(NW, NCHUNK, CHUNK))

    g8 = pl.pallas_call(
        _gate_body,
        grid=(E8 // EB8,),
        in_specs=[
            pl.BlockSpec((EB8, D), lambda e: (e, 0)),
            pl.BlockSpec((D, D), lambda e: (0, 0)),
            pl.BlockSpec((1, 1), lambda e: (0, 0)),
            pl.BlockSpec((D, D), lambda e: (0, 0)),
            pl.BlockSpec((1, D), lambda e: (0, 0)),
        ],
        out_specs=pl.BlockSpec((EB8, D), lambda e: (e, 0)),
        out_shape=jax.ShapeDtypeStruct((E8, D), jnp.float32),
    )(h_i.reshape(E8, D), wk8, b2_i2o[0].reshape(1, 1),
      wg8, jnp.tile(b2_i2o[1:], 8).reshape(1, D))
    g = g8.reshape(E, 16)

    tg = _sc_scatter(g, dst_i.reshape(NW, NCHUNK, CHUNK))

    nb = 2000
    out = pl.pallas_call(
        _final_body,
        grid=(N // nb,),
        in_specs=[
            pl.BlockSpec((NC, nb, 16), lambda n: (0, n, 0)),
            pl.BlockSpec((NC, nb, 16), lambda n: (0, n, 0)),
            pl.BlockSpec((NC, nb, 16), lambda n: (0, n, 0)),
            pl.BlockSpec((NC, nb, 16), lambda n: (0, n, 0)),
            pl.BlockSpec((16, D), lambda n: (0, 0)),
            pl.BlockSpec((1, D), lambda n: (0, 0)),
            pl.BlockSpec((D + HIN, HIN), lambda n: (0, 0)),
            pl.BlockSpec((1, HIN), lambda n: (0, 0)),
            pl.BlockSpec((HIN, D), lambda n: (0, 0)),
            pl.BlockSpec((1, D), lambda n: (0, 0)),
        ],
        out_specs=pl.BlockSpec((nb, D), lambda n: (n, 0)),
        out_shape=jax.ShapeDtypeStruct((N, D), jnp.float32),
    )(sop, dgo, dgi, tg,
      w2_o2i, b2_o2i.reshape(1, D),
      w1_red, b1_red.reshape(1, HIN),
      w2_red, b2_red.reshape(1, D))
    return out


# speedup vs baseline: 10.3042x; 1.0335x over previous
import functools
import jax
import jax.numpy as jnp
from jax import lax
from jax.experimental import pallas as pl
from jax.experimental.pallas import tpu as pltpu
from jax.experimental.pallas import tpu_sc as plsc

N = 10000
E = 320000
D = 128
HIN = 16

NC = 2
NS = 16
NW = NC * NS
EPW = E // NW
CHUNK = 128
NCHUNK = EPW // CHUNK
TAILE = EPW - NCHUNK * CHUNK
G = 6
RPS = 624
TAIL = N - NS * RPS


def _proj_body(nf_ref, wo_ref, wi_ref, ao_ref, bo_ref, ai_ref, bi_ref):
    nf = nf_ref[...]
    ao_ref[...] = jnp.dot(nf, wo_ref[0:D, :],
                          preferred_element_type=jnp.float32)
    bo_ref[...] = jnp.dot(nf, wo_ref[D:2 * D, :],
                          preferred_element_type=jnp.float32)
    ai_ref[...] = jnp.dot(nf, wi_ref[0:D, :],
                          preferred_element_type=jnp.float32)
    bi_ref[...] = jnp.dot(nf, wi_ref[D:2 * D, :],
                          preferred_element_type=jnp.float32)


def _wprep_body(wo_ref, wi_ref, w2i_ref,
                wco8_ref, wci8_ref, wg8_ref, wk8_ref):
    r = lax.broadcasted_iota(jnp.int32, (D, D), 0)
    c = lax.broadcasted_iota(jnp.int32, (D, D), 1)
    mask = (r // 16) == (c // 16)

    def bd(w16):
        return jnp.where(mask, jnp.tile(w16, (8, 8)), 0.0)

    wco8_ref[...] = bd(wo_ref[2 * D:2 * D + 16, :])
    wci8_ref[...] = bd(wi_ref[2 * D:2 * D + 16, :])
    wg8_ref[...] = bd(w2i_ref[:, 1:17])
    wk8_ref[...] = bd(jnp.tile(w2i_ref[:, 0:1], (1, 16)))


def _edge_bias_body(nef_ref, w_ref, b_ref, c_ref):
    c_ref[...] = jnp.dot(nef_ref[...], w_ref[...],
                         preferred_element_type=jnp.float32) + b_ref[...]


def _gate_body(h_ref, wk_ref, bk_ref, wg_ref, bg_ref, g_ref):
    h = h_ref[...]
    m0 = jnp.dot(h, wk_ref[...], preferred_element_type=jnp.float32)
    k = jax.nn.sigmoid(m0 + bk_ref[...])
    g_ref[...] = (jnp.dot(h, wg_ref[...],
                          preferred_element_type=jnp.float32)
                  + bg_ref[...]) * k


def _final_body(sop_ref, dgo_ref, dgi_ref, tg_ref,
                w2o_ref, b2o_ref, w1r_ref, b1r_ref,
                w2r_ref, b2r_ref, out_ref):
    w1a_ref = w1r_ref.at[0:D, :]
    w1b_ref = w1r_ref.at[D:D + 16, :]
    hsum = sop_ref[0] + sop_ref[1]
    dego = dgo_ref[0][:, 0:1] + dgo_ref[1][:, 0:1]
    degi = dgi_ref[0][:, 0:1] + dgi_ref[1][:, 0:1]
    new_nf = (jnp.dot(hsum, w2o_ref[...],
                      preferred_element_type=jnp.float32)
              + dego * b2o_ref[...])
    t = tg_ref[0] + tg_ref[1]
    lane = lax.broadcasted_iota(jnp.int32, t.shape, 1)
    scale = jnp.where(lane < 8, 1.0, 1.0 / jnp.maximum(degi, 1.0))
    ts = t * scale
    hr = (jnp.dot(new_nf, w1a_ref[...], preferred_element_type=jnp.float32)
          + jnp.dot(ts, w1b_ref[...], preferred_element_type=jnp.float32)
          + b1r_ref[...])
    hr = jnp.maximum(hr, 0.2 * hr)
    red = jnp.dot(hr, w2r_ref[...],
                  preferred_element_type=jnp.float32) + b2r_ref[...]
    out_ref[...] = jnp.where(degi > 0, red, new_nf)


def _sc_pass_impl(o2i, a_tab, b_tab, c_hbm, src3, dst3, outs, hout,
                  src_all, dst_all, abufs, bbufs, cbufs, hbufs, dbufs,
                  tidx, ones_buf, stage, accs, gsems, isem):
    cid = lax.axis_index("c")
    sid = lax.axis_index("s")
    wid = sid * NC + cid

    def _zrow(i, _):
        stage[i, :] = jnp.zeros((16,), jnp.float32)
        return 0
    lax.fori_loop(0, RPS, _zrow, 0)
    r0 = sid * RPS
    for acc in accs:
        pltpu.sync_copy(stage, acc.at[pl.ds(r0, RPS)])

    @pl.when(sid == NS - 1)
    def _zero_tail():
        for acc in accs:
            pltpu.sync_copy(stage.at[pl.ds(0, TAIL)],
                            acc.at[pl.ds(NS * RPS, TAIL)])

    onehot = jnp.where(lax.iota(jnp.int32, 16) == 0, 1.0, 0.0)

    def _orow(i, _):
        ones_buf[i, :] = onehot
        return 0
    lax.fori_loop(0, CHUNK, _orow, 0)
    plsc.subcore_barrier()

    base0 = wid * EPW

    cp0 = pltpu.async_copy(src3.at[wid], src_all, isem)
    cp1 = pltpu.async_copy(dst3.at[wid], dst_all, isem)
    cp0.wait()
    cp1.wait()

    def _compute(s, nrows=CHUNK):
        def _row8(i8, _):
            for u in range(8):
                i = i8 * 8 + u
                h = abufs[s][i, :] + bbufs[s][i, :] + cbufs[s][i, :]
                hbufs[s][i, :] = jnp.maximum(h, 0.2 * h)
            return 0
        lax.fori_loop(0, nrows // 8, _row8, 0)

    def _scatter(k, t, nrows=CHUNK, idx=None):
        val = hbufs[k] if nrows == CHUNK else hbufs[k].at[pl.ds(0, nrows)]
        ones = ones_buf if nrows == CHUNK else ones_buf.at[pl.ds(0, nrows)]
        if idx is None:
            idx = dbufs[k]
        if o2i:
            pltpu.sync_copy(val, accs[0].at[idx], add=True)
            pltpu.sync_copy(ones, accs[1].at[idx], add=True)
        else:
            pltpu.sync_copy(
                val, hout.at[pl.ds(base0 + t * CHUNK, nrows)])
            pltpu.sync_copy(ones, accs[0].at[idx], add=True)

    def _group(grp, _):
        t0 = grp * G
        gcps = []
        for k in range(G):
            t = t0 + k
            gcps.append((
                pltpu.async_copy(
                    a_tab.at[src_all.at[pl.ds(t * CHUNK, CHUNK)]],
                    abufs[k], gsems[k]),
                pltpu.async_copy(
                    b_tab.at[dst_all.at[pl.ds(t * CHUNK, CHUNK)]],
                    bbufs[k], gsems[k]),
                pltpu.async_copy(c_hbm.at[pl.ds(base0 + t * CHUNK, CHUNK)],
                                 cbufs[k], gsems[k]),
            ))
        for k in range(G):
            t = t0 + k
            for j in range(CHUNK // 16):
                dbufs[k][pl.ds(j * 16, 16)] = dst_all[
                    pl.ds(t * CHUNK + j * 16, 16)]
            for cp in gcps[k]:
                cp.wait()
            _compute(k)
            _scatter(k, t)
        return 0
    lax.fori_loop(0, NCHUNK // G, _group, 0)

    tb = NCHUNK * CHUNK
    tidx[pl.ds(0, 16)] = dst_all[pl.ds(tb, 16)]
    src_all[pl.ds(0, 16)] = src_all[pl.ds(tb, 16)]
    pltpu.sync_copy(a_tab.at[src_all.at[pl.ds(0, TAILE)]],
                    abufs[0].at[pl.ds(0, TAILE)])
    pltpu.sync_copy(b_tab.at[tidx], bbufs[0].at[pl.ds(0, TAILE)])
    pltpu.sync_copy(c_hbm.at[pl.ds(base0 + tb, TAILE)],
                    cbufs[0].at[pl.ds(0, TAILE)])
    _compute(0, TAILE)
    _scatter(0, NCHUNK, TAILE, idx=tidx)

    plsc.subcore_barrier()

    for acc, out in zip(accs, outs):
        pltpu.sync_copy(acc.at[pl.ds(r0, RPS)], stage)
        pltpu.sync_copy(stage, out.at[cid, pl.ds(r0, RPS)])

    @pl.when(sid == NS - 1)
    def _out_tail():
        tail0 = NS * RPS
        for acc, out in zip(accs, outs):
            pltpu.sync_copy(acc.at[pl.ds(tail0, TAIL)],
                            stage.at[pl.ds(0, TAIL)])
            pltpu.sync_copy(stage.at[pl.ds(0, TAIL)],
                            out.at[cid, pl.ds(tail0, TAIL)])


def _sc_o2i_body(ao, bo, co, src3, dst3, sop, dgo,
                 src_all, dst_all,
                 *rest):
    bufs = rest[:4 * G]
    d = rest[4 * G:5 * G]
    tidx, ones_buf, stage, acc_h, acc_d = rest[5 * G:5 * G + 5]
    gsems = rest[5 * G + 5:6 * G + 5]
    isem = rest[6 * G + 5]
    _sc_pass_impl(True, ao, bo, co, src3, dst3, (sop, dgo), None,
                  src_all, dst_all,
                  bufs[0:G], bufs[G:2 * G], bufs[2 * G:3 * G],
                  bufs[3 * G:4 * G], d, tidx, ones_buf, stage,
                  (acc_h, acc_d), gsems, isem)


def _sc_i2o_body(ai, bi, ci, src3, dst3, dgi, hout,
                 src_all, dst_all,
                 *rest):
    bufs = rest[:4 * G]
    d = rest[4 * G:5 * G]
    tidx, ones_buf, stage, acc_d = rest[5 * G:5 * G + 4]
    gsems = rest[5 * G + 4:6 * G + 4]
    isem = rest[6 * G + 4]
    _sc_pass_impl(False, ai, bi, ci, src3, dst3, (dgi,), hout,
                  src_all, dst_all,
                  bufs[0:G], bufs[G:2 * G], bufs[2 * G:3 * G],
                  bufs[3 * G:4 * G], d, tidx, ones_buf, stage, (acc_d,),
                  gsems, isem)


def _sc_scatter_body(g_hbm, dst3, tg, dst_all, *rest):
    gbufs = rest[0:G]
    dbufs = rest[G:2 * G]
    tidx, stage, t_acc = rest[2 * G:2 * G + 3]
    gsems = rest[2 * G + 3:3 * G + 3]
    isem = rest[3 * G + 3]
    cid = lax.axis_index("c")
    sid = lax.axis_index("s")
    wid = sid * NC + cid

    def _zrow(i, _):
        stage[i, :] = jnp.zeros((16,), jnp.float32)
        return 0
    lax.fori_loop(0, RPS, _zrow, 0)
    r0 = sid * RPS
    pltpu.sync_copy(stage, t_acc.at[pl.ds(r0, RPS)])

    @pl.when(sid == NS - 1)
    def _zero_tail():
        pltpu.sync_copy(stage.at[pl.ds(0, TAIL)],
                        t_acc.at[pl.ds(NS * RPS, TAIL)])
    plsc.subcore_barrier()

    base0 = wid * EPW
    cp = pltpu.async_copy(dst3.at[wid], dst_all, isem)
    cp.wait()

    def _group(grp, _):
        t0 = grp * G
        gcps = []
        for k in range(G):
            t = t0 + k
            gcps.append(pltpu.async_copy(
                g_hbm.at[pl.ds(base0 + t * CHUNK, CHUNK)], gbufs[k],
                gsems[k]))
        for k in range(G):
            t = t0 + k
            for j in range(CHUNK // 16):
                dbufs[k][pl.ds(j * 16, 16)] = dst_all[
                    pl.ds(t * CHUNK + j * 16, 16)]
            gcps[k].wait()
            pltpu.sync_copy(gbufs[k], t_acc.at[dbufs[k]], add=True)
        return 0
    lax.fori_loop(0, NCHUNK // G, _group, 0)

    tb = NCHUNK * CHUNK
    tidx[pl.ds(0, 16)] = dst_all[pl.ds(tb, 16)]
    pltpu.sync_copy(g_hbm.at[pl.ds(base0 + tb, TAILE)],
                    gbufs[0].at[pl.ds(0, TAILE)])
    pltpu.sync_copy(gbufs[0].at[pl.ds(0, TAILE)],
                    t_acc.at[tidx], add=True)

    plsc.subcore_barrier()
    pltpu.sync_copy(t_acc.at[pl.ds(r0, RPS)], stage)
    pltpu.sync_copy(stage, tg.at[cid, pl.ds(r0, RPS)])

    @pl.when(sid == NS - 1)
    def _out_tail():
        tail0 = NS * RPS
        pltpu.sync_copy(t_acc.at[pl.ds(tail0, TAIL)], stage.at[pl.ds(0, TAIL)])
        pltpu.sync_copy(stage.at[pl.ds(0, TAIL)], tg.at[cid, pl.ds(tail0, TAIL)])


_SC_MESH = plsc.VectorSubcoreMesh(core_axis_name="c", subcore_axis_name="s")
_SC_PARAMS = pltpu.CompilerParams(use_tc_tiling_on_sc=False)

_EDGE_SCRATCH = (
    [pltpu.VMEM((EPW,), jnp.int32)] * 2
    + [pltpu.VMEM((CHUNK, 16), jnp.float32)] * (4 * G)
    + [pltpu.VMEM((CHUNK,), jnp.int32)] * G
    + [pltpu.VMEM((TAILE,), jnp.int32)]
    + [pltpu.VMEM((CHUNK, 16), jnp.float32)]
    + [pltpu.VMEM((RPS, 16), jnp.float32)]
)

_sc_o2i = pl.kernel(
    _sc_o2i_body,
    out_type=(
        jax.ShapeDtypeStruct((NC, N, 16), jnp.float32),
        jax.ShapeDtypeStruct((NC, N, 16), jnp.float32),
    ),
    mesh=_SC_MESH,
    scratch_types=(
        _EDGE_SCRATCH
        + [pltpu.VMEM_SHARED((N, 16), jnp.float32)] * 2
        + [pltpu.SemaphoreType.DMA] * (G + 1)
    ),
    compiler_params=_SC_PARAMS,
)

_sc_i2o = pl.kernel(
    _sc_i2o_body,
    out_type=(
        jax.ShapeDtypeStruct((NC, N, 16), jnp.float32),
        jax.ShapeDtypeStruct((E, 16), jnp.float32),
    ),
    mesh=_SC_MESH,
    scratch_types=(
        _EDGE_SCRATCH
        + [pltpu.VMEM_SHARED((N, 16), jnp.float32)]
        + [pltpu.SemaphoreType.DMA] * (G + 1)
    ),
    compiler_params=_SC_PARAMS,
)

_sc_scatter = pl.kernel(
    _sc_scatter_body,
    out_type=jax.ShapeDtypeStruct((NC, N, 16), jnp.float32),
    mesh=_SC_MESH,
    scratch_types=(
        [pltpu.VMEM((EPW,), jnp.int32)]
        + [pltpu.VMEM((CHUNK, 16), jnp.float32)] * G
        + [pltpu.VMEM((CHUNK,), jnp.int32)] * G
        + [pltpu.VMEM((TAILE,), jnp.int32)]
        + [pltpu.VMEM((RPS, 16), jnp.float32)]
        + [pltpu.VMEM_SHARED((N, 16), jnp.float32)]
        + [pltpu.SemaphoreType.DMA] * (G + 1)
    ),
    compiler_params=_SC_PARAMS,
)


E8 = E // 8
EB8 = 5000


@jax.jit
def kernel(nf, edge_index_out, nef_out, edge_index_in, nef_in,
           w1_o2i, b1_o2i, w2_o2i, b2_o2i,
           w1_i2o, b1_i2o, w2_i2o, b2_i2o,
           w1_red, b1_red, w2_red, b2_red):
    nf = nf.astype(jnp.float32)
    src_o, dst_o = edge_index_out[0], edge_index_out[1]
    src_i, dst_i = edge_index_in[0], edge_index_in[1]

    ao, bo, ai, bi = pl.pallas_call(
        _proj_body,
        out_shape=[jax.ShapeDtypeStruct((N, 16), jnp.float32)] * 4,
    )(nf, w1_o2i, w1_i2o)

    wco8, wci8, wg8, wk8 = pl.pallas_call(
        _wprep_body,
        out_shape=[jax.ShapeDtypeStruct((D, D), jnp.float32)] * 4,
    )(w1_o2i, w1_i2o, w2_i2o)

    def _edge_bias(nef, w8, b16):
        c8 = pl.pallas_call(
            _edge_bias_body,
            grid=(E8 // EB8,),
            in_specs=[
                pl.BlockSpec((EB8, D), lambda e: (e, 0)),
                pl.BlockSpec((D, D), lambda e: (0, 0)),
                pl.BlockSpec((1, D), lambda e: (0, 0)),
            ],
            out_specs=pl.BlockSpec((EB8, D), lambda e: (e, 0)),
            out_shape=jax.ShapeDtypeStruct((E8, D), jnp.float32),
        )(nef.reshape(E8, D), w8, jnp.tile(b16, 8).reshape(1, D))
        return c8.reshape(E, 16)

    ci = _edge_bias(nef_in, wci8, b1_i2o)
    co = _edge_bias(nef_out, wco8, b1_o2i)

    dgi, h_i = _sc_i2o(
        ai, bi, ci,
        src_i.reshape(NW, EPW), dst_i.reshape(NW, EPW))
    sop, dgo = _sc_o2i(
        ao, bo, co,
        src_o.reshape(NW, EPW), dst_o.reshape(NW, EPW))

    g8 = pl.pallas_call(
        _gate_body,
        grid=(E8 // EB8,),
        in_specs=[
            pl.BlockSpec((EB8, D), lambda e: (e, 0)),
            pl.BlockSpec((D, D), lambda e: (0, 0)),
            pl.BlockSpec((1, 1), lambda e: (0, 0)),
            pl.BlockSpec((D, D), lambda e: (0, 0)),
            pl.BlockSpec((1, D), lambda e: (0, 0)),
        ],
        out_specs=pl.BlockSpec((EB8, D), lambda e: (e, 0)),
        out_shape=jax.ShapeDtypeStruct((E8, D), jnp.float32),
    )(h_i.reshape(E8, D), wk8, b2_i2o[0].reshape(1, 1),
      wg8, jnp.tile(b2_i2o[1:], 8).reshape(1, D))
    g = g8.reshape(E, 16)

    tg = _sc_scatter(g, dst_i.reshape(NW, EPW))

    nb = 2000
    out = pl.pallas_call(
        _final_body,
        grid=(N // nb,),
        in_specs=[
            pl.BlockSpec((NC, nb, 16), lambda n: (0, n, 0)),
            pl.BlockSpec((NC, nb, 16), lambda n: (0, n, 0)),
            pl.BlockSpec((NC, nb, 16), lambda n: (0, n, 0)),
            pl.BlockSpec((NC, nb, 16), lambda n: (0, n, 0)),
            pl.BlockSpec((16, D), lambda n: (0, 0)),
            pl.BlockSpec((1, D), lambda n: (0, 0)),
            pl.BlockSpec((D + HIN, HIN), lambda n: (0, 0)),
            pl.BlockSpec((1, HIN), lambda n: (0, 0)),
            pl.BlockSpec((HIN, D), lambda n: (0, 0)),
            pl.BlockSpec((1, D), lambda n: (0, 0)),
        ],
        out_specs=pl.BlockSpec((nb, D), lambda n: (n, 0)),
        out_shape=jax.ShapeDtypeStruct((N, D), jnp.float32),
    )(sop, dgo, dgi, tg,
      w2_o2i, b2_o2i.reshape(1, D),
      w1_red, b1_red.reshape(1, HIN),
      w2_red, b2_red.reshape(1, D))
    return out


# speedup vs baseline: 10.5888x; 1.0276x over previous
import functools
import jax
import jax.numpy as jnp
from jax import lax
from jax.experimental import pallas as pl
from jax.experimental.pallas import tpu as pltpu
from jax.experimental.pallas import tpu_sc as plsc

N = 10000
E = 320000
D = 128
HIN = 16

NC = 2
NS = 16
NW = NC * NS
EPW = E // NW
CHUNK = 128
NCHUNK = EPW // CHUNK
TAILE = EPW - NCHUNK * CHUNK
G = 6
RPS = 624
TAIL = N - NS * RPS


def _proj_body(nf_ref, wo_ref, wi_ref, ao_ref, bo_ref, ai_ref, bi_ref):
    nf = nf_ref[...]
    ao_ref[...] = jnp.dot(nf, wo_ref[0:D, :],
                          preferred_element_type=jnp.float32)
    bo_ref[...] = jnp.dot(nf, wo_ref[D:2 * D, :],
                          preferred_element_type=jnp.float32)
    ai_ref[...] = jnp.dot(nf, wi_ref[0:D, :],
                          preferred_element_type=jnp.float32)
    bi_ref[...] = jnp.dot(nf, wi_ref[D:2 * D, :],
                          preferred_element_type=jnp.float32)


def _wprep_body(wo_ref, wi_ref, w2i_ref,
                wco8_ref, wci8_ref, wg8_ref, wk8_ref):
    r = lax.broadcasted_iota(jnp.int32, (D, D), 0)
    c = lax.broadcasted_iota(jnp.int32, (D, D), 1)
    mask = (r // 16) == (c // 16)

    def bd(w16):
        return jnp.where(mask, jnp.tile(w16, (8, 8)), 0.0)

    wco8_ref[...] = bd(wo_ref[2 * D:2 * D + 16, :])
    wci8_ref[...] = bd(wi_ref[2 * D:2 * D + 16, :])
    wg8_ref[...] = bd(w2i_ref[:, 1:17])
    wk8_ref[...] = bd(jnp.tile(w2i_ref[:, 0:1], (1, 16)))


def _edge_bias_body(nef_ref, w_ref, b_ref, c_ref):
    c_ref[...] = jnp.dot(nef_ref[...], w_ref[...],
                         preferred_element_type=jnp.float32) + b_ref[...]


def _gate_body(h_ref, wk_ref, bk_ref, wg_ref, bg_ref, g_ref):
    h = h_ref[...]
    m0 = jnp.dot(h, wk_ref[...], preferred_element_type=jnp.float32)
    k = jax.nn.sigmoid(m0 + bk_ref[...])
    g_ref[...] = (jnp.dot(h, wg_ref[...],
                          preferred_element_type=jnp.float32)
                  + bg_ref[...]) * k


def _final_body(sop_ref, dgo_ref, dgi_ref, tg_ref,
                w2o_ref, b2o_ref, w1r_ref, b1r_ref,
                w2r_ref, b2r_ref, out_ref):
    w1a_ref = w1r_ref.at[0:D, :]
    w1b_ref = w1r_ref.at[D:D + 16, :]
    hsum = sop_ref[0] + sop_ref[1]
    dego = dgo_ref[0][:, 0:1] + dgo_ref[1][:, 0:1]
    degi = dgi_ref[0][:, 0:1] + dgi_ref[1][:, 0:1]
    new_nf = (jnp.dot(hsum, w2o_ref[...],
                      preferred_element_type=jnp.float32)
              + dego * b2o_ref[...])
    t = tg_ref[0] + tg_ref[1]
    lane = lax.broadcasted_iota(jnp.int32, t.shape, 1)
    scale = jnp.where(lane < 8, 1.0, 1.0 / jnp.maximum(degi, 1.0))
    ts = t * scale
    hr = (jnp.dot(new_nf, w1a_ref[...], preferred_element_type=jnp.float32)
          + jnp.dot(ts, w1b_ref[...], preferred_element_type=jnp.float32)
          + b1r_ref[...])
    hr = jnp.maximum(hr, 0.2 * hr)
    red = jnp.dot(hr, w2r_ref[...],
                  preferred_element_type=jnp.float32) + b2r_ref[...]
    out_ref[...] = jnp.where(degi > 0, red, new_nf)


def _sc_pass_impl(o2i, a_tab, b_tab, c_hbm, ei3, outs, hout,
                  src_all, dst_all, abufs, bbufs, cbufs, hbufs, dbufs,
                  tidx, ones_buf, stage, accs, gsems, isem):
    cid = lax.axis_index("c")
    sid = lax.axis_index("s")
    wid = sid * NC + cid

    def _zrow(i, _):
        stage[i, :] = jnp.zeros((16,), jnp.float32)
        return 0
    lax.fori_loop(0, RPS, _zrow, 0)
    r0 = sid * RPS
    for acc in accs:
        pltpu.sync_copy(stage, acc.at[pl.ds(r0, RPS)])

    @pl.when(sid == NS - 1)
    def _zero_tail():
        for acc in accs:
            pltpu.sync_copy(stage.at[pl.ds(0, TAIL)],
                            acc.at[pl.ds(NS * RPS, TAIL)])

    onehot = jnp.where(lax.iota(jnp.int32, 16) == 0, 1.0, 0.0)

    def _orow(i, _):
        ones_buf[i, :] = onehot
        return 0
    lax.fori_loop(0, CHUNK, _orow, 0)
    plsc.subcore_barrier()

    base0 = wid * EPW

    cp0 = pltpu.async_copy(ei3.at[0, wid], src_all, isem)
    cp1 = pltpu.async_copy(ei3.at[1, wid], dst_all, isem)
    cp0.wait()
    cp1.wait()

    def _compute(s, nrows=CHUNK):
        def _row8(i8, _):
            for u in range(8):
                i = i8 * 8 + u
                h = abufs[s][i, :] + bbufs[s][i, :] + cbufs[s][i, :]
                hbufs[s][i, :] = jnp.maximum(h, 0.2 * h)
            return 0
        lax.fori_loop(0, nrows // 8, _row8, 0)

    def _scatter(k, t, nrows=CHUNK, idx=None):
        val = hbufs[k] if nrows == CHUNK else hbufs[k].at[pl.ds(0, nrows)]
        ones = ones_buf if nrows == CHUNK else ones_buf.at[pl.ds(0, nrows)]
        if idx is None:
            idx = dbufs[k]
        if o2i:
            pltpu.sync_copy(val, accs[0].at[idx], add=True)
            pltpu.sync_copy(ones, accs[1].at[idx], add=True)
        else:
            pltpu.sync_copy(
                val, hout.at[pl.ds(base0 + t * CHUNK, nrows)])
            pltpu.sync_copy(ones, accs[0].at[idx], add=True)

    def _group(grp, _):
        t0 = grp * G
        gcps = []
        for k in range(G):
            t = t0 + k
            gcps.append((
                pltpu.async_copy(
                    a_tab.at[src_all.at[pl.ds(t * CHUNK, CHUNK)]],
                    abufs[k], gsems[k]),
                pltpu.async_copy(
                    b_tab.at[dst_all.at[pl.ds(t * CHUNK, CHUNK)]],
                    bbufs[k], gsems[k]),
                pltpu.async_copy(c_hbm.at[pl.ds(base0 + t * CHUNK, CHUNK)],
                                 cbufs[k], gsems[k]),
            ))
        for k in range(G):
            t = t0 + k
            for j in range(CHUNK // 16):
                dbufs[k][pl.ds(j * 16, 16)] = dst_all[
                    pl.ds(t * CHUNK + j * 16, 16)]
            for cp in gcps[k]:
                cp.wait()
            _compute(k)
            _scatter(k, t)
        return 0
    lax.fori_loop(0, NCHUNK // G, _group, 0)

    tb = NCHUNK * CHUNK
    tidx[pl.ds(0, 16)] = dst_all[pl.ds(tb, 16)]
    src_all[pl.ds(0, 16)] = src_all[pl.ds(tb, 16)]
    pltpu.sync_copy(a_tab.at[src_all.at[pl.ds(0, TAILE)]],
                    abufs[0].at[pl.ds(0, TAILE)])
    pltpu.sync_copy(b_tab.at[tidx], bbufs[0].at[pl.ds(0, TAILE)])
    pltpu.sync_copy(c_hbm.at[pl.ds(base0 + tb, TAILE)],
                    cbufs[0].at[pl.ds(0, TAILE)])
    _compute(0, TAILE)
    _scatter(0, NCHUNK, TAILE, idx=tidx)

    plsc.subcore_barrier()

    for acc, out in zip(accs, outs):
        pltpu.sync_copy(acc.at[pl.ds(r0, RPS)], stage)
        pltpu.sync_copy(stage, out.at[cid, pl.ds(r0, RPS)])

    @pl.when(sid == NS - 1)
    def _out_tail():
        tail0 = NS * RPS
        for acc, out in zip(accs, outs):
            pltpu.sync_copy(acc.at[pl.ds(tail0, TAIL)],
                            stage.at[pl.ds(0, TAIL)])
            pltpu.sync_copy(stage.at[pl.ds(0, TAIL)],
                            out.at[cid, pl.ds(tail0, TAIL)])


def _sc_o2i_body(ao, bo, co, ei3, sop, dgo,
                 src_all, dst_all,
                 *rest):
    bufs = rest[:4 * G]
    d = rest[4 * G:5 * G]
    tidx, ones_buf, stage, acc_h, acc_d = rest[5 * G:5 * G + 5]
    gsems = rest[5 * G + 5:6 * G + 5]
    isem = rest[6 * G + 5]
    _sc_pass_impl(True, ao, bo, co, ei3, (sop, dgo), None,
                  src_all, dst_all,
                  bufs[0:G], bufs[G:2 * G], bufs[2 * G:3 * G],
                  bufs[3 * G:4 * G], d, tidx, ones_buf, stage,
                  (acc_h, acc_d), gsems, isem)


def _sc_i2o_body(ai, bi, ci, ei3, dgi, hout,
                 src_all, dst_all,
                 *rest):
    bufs = rest[:4 * G]
    d = rest[4 * G:5 * G]
    tidx, ones_buf, stage, acc_d = rest[5 * G:5 * G + 4]
    gsems = rest[5 * G + 4:6 * G + 4]
    isem = rest[6 * G + 4]
    _sc_pass_impl(False, ai, bi, ci, ei3, (dgi,), hout,
                  src_all, dst_all,
                  bufs[0:G], bufs[G:2 * G], bufs[2 * G:3 * G],
                  bufs[3 * G:4 * G], d, tidx, ones_buf, stage, (acc_d,),
                  gsems, isem)


def _sc_scatter_body(g_hbm, ei3, tg, dst_all, *rest):
    gbufs = rest[0:G]
    dbufs = rest[G:2 * G]
    tidx, stage, t_acc = rest[2 * G:2 * G + 3]
    gsems = rest[2 * G + 3:3 * G + 3]
    isem = rest[3 * G + 3]
    cid = lax.axis_index("c")
    sid = lax.axis_index("s")
    wid = sid * NC + cid

    def _zrow(i, _):
        stage[i, :] = jnp.zeros((16,), jnp.float32)
        return 0
    lax.fori_loop(0, RPS, _zrow, 0)
    r0 = sid * RPS
    pltpu.sync_copy(stage, t_acc.at[pl.ds(r0, RPS)])

    @pl.when(sid == NS - 1)
    def _zero_tail():
        pltpu.sync_copy(stage.at[pl.ds(0, TAIL)],
                        t_acc.at[pl.ds(NS * RPS, TAIL)])
    plsc.subcore_barrier()

    base0 = wid * EPW
    cp = pltpu.async_copy(ei3.at[1, wid], dst_all, isem)
    cp.wait()

    def _group(grp, _):
        t0 = grp * G
        gcps = []
        for k in range(G):
            t = t0 + k
            gcps.append(pltpu.async_copy(
                g_hbm.at[pl.ds(base0 + t * CHUNK, CHUNK)], gbufs[k],
                gsems[k]))
        for k in range(G):
            t = t0 + k
            for j in range(CHUNK // 16):
                dbufs[k][pl.ds(j * 16, 16)] = dst_all[
                    pl.ds(t * CHUNK + j * 16, 16)]
            gcps[k].wait()
            pltpu.sync_copy(gbufs[k], t_acc.at[dbufs[k]], add=True)
        return 0
    lax.fori_loop(0, NCHUNK // G, _group, 0)

    tb = NCHUNK * CHUNK
    tidx[pl.ds(0, 16)] = dst_all[pl.ds(tb, 16)]
    pltpu.sync_copy(g_hbm.at[pl.ds(base0 + tb, TAILE)],
                    gbufs[0].at[pl.ds(0, TAILE)])
    pltpu.sync_copy(gbufs[0].at[pl.ds(0, TAILE)],
                    t_acc.at[tidx], add=True)

    plsc.subcore_barrier()
    pltpu.sync_copy(t_acc.at[pl.ds(r0, RPS)], stage)
    pltpu.sync_copy(stage, tg.at[cid, pl.ds(r0, RPS)])

    @pl.when(sid == NS - 1)
    def _out_tail():
        tail0 = NS * RPS
        pltpu.sync_copy(t_acc.at[pl.ds(tail0, TAIL)], stage.at[pl.ds(0, TAIL)])
        pltpu.sync_copy(stage.at[pl.ds(0, TAIL)], tg.at[cid, pl.ds(tail0, TAIL)])


_SC_MESH = plsc.VectorSubcoreMesh(core_axis_name="c", subcore_axis_name="s")
_SC_PARAMS = pltpu.CompilerParams(use_tc_tiling_on_sc=False)

_EDGE_SCRATCH = (
    [pltpu.VMEM((EPW,), jnp.int32)] * 2
    + [pltpu.VMEM((CHUNK, 16), jnp.float32)] * (4 * G)
    + [pltpu.VMEM((CHUNK,), jnp.int32)] * G
    + [pltpu.VMEM((TAILE,), jnp.int32)]
    + [pltpu.VMEM((CHUNK, 16), jnp.float32)]
    + [pltpu.VMEM((RPS, 16), jnp.float32)]
)

_sc_o2i = pl.kernel(
    _sc_o2i_body,
    out_type=(
        jax.ShapeDtypeStruct((NC, N, 16), jnp.float32),
        jax.ShapeDtypeStruct((NC, N, 16), jnp.float32),
    ),
    mesh=_SC_MESH,
    scratch_types=(
        _EDGE_SCRATCH
        + [pltpu.VMEM_SHARED((N, 16), jnp.float32)] * 2
        + [pltpu.SemaphoreType.DMA] * (G + 1)
    ),
    compiler_params=_SC_PARAMS,
)

_sc_i2o = pl.kernel(
    _sc_i2o_body,
    out_type=(
        jax.ShapeDtypeStruct((NC, N, 16), jnp.float32),
        jax.ShapeDtypeStruct((E, 16), jnp.float32),
    ),
    mesh=_SC_MESH,
    scratch_types=(
        _EDGE_SCRATCH
        + [pltpu.VMEM_SHARED((N, 16), jnp.float32)]
        + [pltpu.SemaphoreType.DMA] * (G + 1)
    ),
    compiler_params=_SC_PARAMS,
)

_sc_scatter = pl.kernel(
    _sc_scatter_body,
    out_type=jax.ShapeDtypeStruct((NC, N, 16), jnp.float32),
    mesh=_SC_MESH,
    scratch_types=(
        [pltpu.VMEM((EPW,), jnp.int32)]
        + [pltpu.VMEM((CHUNK, 16), jnp.float32)] * G
        + [pltpu.VMEM((CHUNK,), jnp.int32)] * G
        + [pltpu.VMEM((TAILE,), jnp.int32)]
        + [pltpu.VMEM((RPS, 16), jnp.float32)]
        + [pltpu.VMEM_SHARED((N, 16), jnp.float32)]
        + [pltpu.SemaphoreType.DMA] * (G + 1)
    ),
    compiler_params=_SC_PARAMS,
)


E8 = E // 8
EB8 = 5000


@jax.jit
def kernel(nf, edge_index_out, nef_out, edge_index_in, nef_in,
           w1_o2i, b1_o2i, w2_o2i, b2_o2i,
           w1_i2o, b1_i2o, w2_i2o, b2_i2o,
           w1_red, b1_red, w2_red, b2_red):
    nf = nf.astype(jnp.float32)
    ei_o = edge_index_out.reshape(2, NW, EPW)
    ei_i = edge_index_in.reshape(2, NW, EPW)

    ao, bo, ai, bi = pl.pallas_call(
        _proj_body,
        out_shape=[jax.ShapeDtypeStruct((N, 16), jnp.float32)] * 4,
    )(nf, w1_o2i, w1_i2o)

    wco8, wci8, wg8, wk8 = pl.pallas_call(
        _wprep_body,
        out_shape=[jax.ShapeDtypeStruct((D, D), jnp.float32)] * 4,
    )(w1_o2i, w1_i2o, w2_i2o)

    def _edge_bias(nef, w8, b16):
        c8 = pl.pallas_call(
            _edge_bias_body,
            grid=(E8 // EB8,),
            in_specs=[
                pl.BlockSpec((EB8, D), lambda e: (e, 0)),
                pl.BlockSpec((D, D), lambda e: (0, 0)),
                pl.BlockSpec((1, D), lambda e: (0, 0)),
            ],
            out_specs=pl.BlockSpec((EB8, D), lambda e: (e, 0)),
            out_shape=jax.ShapeDtypeStruct((E8, D), jnp.float32),
        )(nef.reshape(E8, D), w8, jnp.tile(b16, 8).reshape(1, D))
        return c8.reshape(E, 16)

    ci = _edge_bias(nef_in, wci8, b1_i2o)
    co = _edge_bias(nef_out, wco8, b1_o2i)

    dgi, h_i = _sc_i2o(ai, bi, ci, ei_i)
    sop, dgo = _sc_o2i(ao, bo, co, ei_o)

    g8 = pl.pallas_call(
        _gate_body,
        grid=(E8 // EB8,),
        in_specs=[
            pl.BlockSpec((EB8, D), lambda e: (e, 0)),
            pl.BlockSpec((D, D), lambda e: (0, 0)),
            pl.BlockSpec((1, 1), lambda e: (0, 0)),
            pl.BlockSpec((D, D), lambda e: (0, 0)),
            pl.BlockSpec((1, D), lambda e: (0, 0)),
        ],
        out_specs=pl.BlockSpec((EB8, D), lambda e: (e, 0)),
        out_shape=jax.ShapeDtypeStruct((E8, D), jnp.float32),
    )(h_i.reshape(E8, D), wk8, b2_i2o[0].reshape(1, 1),
      wg8, jnp.tile(b2_i2o[1:], 8).reshape(1, D))
    g = g8.reshape(E, 16)

    tg = _sc_scatter(g, ei_i)

    nb = 2000
    out = pl.pallas_call(
        _final_body,
        grid=(N // nb,),
        in_specs=[
            pl.BlockSpec((NC, nb, 16), lambda n: (0, n, 0)),
            pl.BlockSpec((NC, nb, 16), lambda n: (0, n, 0)),
            pl.BlockSpec((NC, nb, 16), lambda n: (0, n, 0)),
            pl.BlockSpec((NC, nb, 16), lambda n: (0, n, 0)),
            pl.BlockSpec((16, D), lambda n: (0, 0)),
            pl.BlockSpec((1, D), lambda n: (0, 0)),
            pl.BlockSpec((D + HIN, HIN), lambda n: (0, 0)),
            pl.BlockSpec((1, HIN), lambda n: (0, 0)),
            pl.BlockSpec((HIN, D), lambda n: (0, 0)),
            pl.BlockSpec((1, D), lambda n: (0, 0)),
        ],
        out_specs=pl.BlockSpec((nb, D), lambda n: (n, 0)),
        out_shape=jax.ShapeDtypeStruct((N, D), jnp.float32),
    )(sop, dgo, dgi, tg,
      w2_o2i, b2_o2i.reshape(1, D),
      w1_red, b1_red.reshape(1, HIN),
      w2_red, b2_red.reshape(1, D))
    return out
